# Initial kernel scaffold; baseline (speedup 1.0000x reference)
import jax
import jax.numpy as jnp
from jax.experimental import pallas as pl


def kernel(x, pos, edge_index, W_e1, b_e1, W_e2, b_e2, W_n, b_n, W_c, b_c):
    raise NotImplementedError("write your pallas kernel here")


# speedup vs baseline: 3.0073x; 3.0073x over previous
import jax
import jax.numpy as jnp
from jax import lax
from jax.experimental import pallas as pl
from jax.experimental.pallas import tpu as pltpu
from jax.experimental.pallas import tpu_sc as plsc

N_NODES = 10000
N_EDGES = 320000
FEAT = 128
HID = 128
N_PAD = 10240
E_PAD = 327680
NW = 32
CHUNK = 128
EDGES_PER_WORKER = E_PAD // NW
CHUNKS_PER_WORKER = EDGES_PER_WORKER // CHUNK
ROWS_PER_TILE = N_PAD // 16
L = 16


def _stage1_body(x_ref, wa_ref, wb_ref, wnx_ref, t_ref, u_ref, xc_ref):
    x = x_ref[...]
    t_ref[...] = jnp.dot(x, wa_ref[...], preferred_element_type=jnp.float32)
    u_ref[...] = jnp.dot(x, wb_ref[...], preferred_element_type=jnp.float32)
    xc_ref[...] = jnp.dot(x, wnx_ref[...], preferred_element_type=jnp.float32)


def _stage1(x_pad, wa_t, wb_t, wnx_t):
    blk = 1024
    return pl.pallas_call(
        _stage1_body,
        grid=(N_PAD // blk,),
        in_specs=[
            pl.BlockSpec((blk, FEAT), lambda i: (i, 0)),
            pl.BlockSpec((FEAT, FEAT), lambda i: (0, 0)),
            pl.BlockSpec((FEAT, FEAT), lambda i: (0, 0)),
            pl.BlockSpec((FEAT, FEAT), lambda i: (0, 0)),
        ],
        out_specs=[
            pl.BlockSpec((blk, FEAT), lambda i: (i, 0)),
            pl.BlockSpec((blk, FEAT), lambda i: (i, 0)),
            pl.BlockSpec((blk, FEAT), lambda i: (i, 0)),
        ],
        out_shape=[
            jax.ShapeDtypeStruct((N_PAD, FEAT), jnp.float32),
            jax.ShapeDtypeStruct((N_PAD, FEAT), jnp.float32),
            jax.ShapeDtypeStruct((N_PAD, FEAT), jnp.float32),
        ],
    )(x_pad, wa_t, wb_t, wnx_t)


def _gather_body(t_hbm, u_hbm, row_hbm, col_hbm, px_hbm, py_hbm, pz_hbm,
                 p_hbm, pdx_hbm, pdy_hbm, pdz_hbm, d2_hbm,
                 idx_r, idx_c, buf_t, buf_u, posx, posy, posz,
                 pdx_b, pdy_b, pdz_b, d2_b, sem_t, sem_u):
    c = lax.axis_index("c")
    s = lax.axis_index("s")
    wid = c * 16 + s

    pltpu.sync_copy(px_hbm, posx)
    pltpu.sync_copy(py_hbm, posy)
    pltpu.sync_copy(pz_hbm, posz)

    def chunk(g, carry):
        base = pl.multiple_of(wid * EDGES_PER_WORKER + g * CHUNK, CHUNK)
        pltpu.sync_copy(row_hbm.at[pl.ds(base, CHUNK)], idx_r)
        pltpu.sync_copy(col_hbm.at[pl.ds(base, CHUNK)], idx_c)
        cp_t = pltpu.async_copy(t_hbm.at[idx_r], buf_t, sem_t)
        cp_u = pltpu.async_copy(u_hbm.at[idx_c], buf_u, sem_u)

        for k in range(CHUNK // L):
            sl = pl.ds(k * L, L)
            ir = idx_r[sl]
            ic = idx_c[sl]
            dx = plsc.load_gather(posx, [ir]) - plsc.load_gather(posx, [ic])
            dy = plsc.load_gather(posy, [ir]) - plsc.load_gather(posy, [ic])
            dz = plsc.load_gather(posz, [ir]) - plsc.load_gather(posz, [ic])
            pdx_b[sl] = dx
            pdy_b[sl] = dy
            pdz_b[sl] = dz
            d2_b[sl] = dx * dx + dy * dy + dz * dz

        pltpu.sync_copy(pdx_b, pdx_hbm.at[pl.ds(base, CHUNK)])
        pltpu.sync_copy(pdy_b, pdy_hbm.at[pl.ds(base, CHUNK)])
        pltpu.sync_copy(pdz_b, pdz_hbm.at[pl.ds(base, CHUNK)])
        pltpu.sync_copy(d2_b, d2_hbm.at[pl.ds(base, CHUNK)])

        cp_t.wait()
        cp_u.wait()

        def add_row(i, carry2):
            for j in range(FEAT // L):
                sl2 = pl.ds(j * L, L)
                buf_t[i, sl2] = buf_t[i, sl2] + buf_u[i, sl2]
            return carry2

        lax.fori_loop(0, CHUNK, add_row, 0, unroll=False)
        pltpu.sync_copy(buf_t, p_hbm.at[pl.ds(base, CHUNK)])
        return carry

    lax.fori_loop(0, CHUNKS_PER_WORKER, chunk, 0, unroll=False)


def _stage2(t_tab, u_tab, row_idx, col_idx, px, py, pz):
    mesh = plsc.VectorSubcoreMesh(core_axis_name="c", subcore_axis_name="s")
    f = pl.kernel(
        _gather_body,
        compiler_params=pltpu.CompilerParams(needs_layout_passes=False),
        out_type=[
            jax.ShapeDtypeStruct((E_PAD, FEAT), jnp.float32),
            jax.ShapeDtypeStruct((E_PAD,), jnp.float32),
            jax.ShapeDtypeStruct((E_PAD,), jnp.float32),
            jax.ShapeDtypeStruct((E_PAD,), jnp.float32),
            jax.ShapeDtypeStruct((E_PAD,), jnp.float32),
        ],
        mesh=mesh,
        scratch_types=[
            pltpu.VMEM((CHUNK,), jnp.int32),
            pltpu.VMEM((CHUNK,), jnp.int32),
            pltpu.VMEM((CHUNK, FEAT), jnp.float32),
            pltpu.VMEM((CHUNK, FEAT), jnp.float32),
            pltpu.VMEM((N_PAD,), jnp.float32),
            pltpu.VMEM((N_PAD,), jnp.float32),
            pltpu.VMEM((N_PAD,), jnp.float32),
            pltpu.VMEM((CHUNK,), jnp.float32),
            pltpu.VMEM((CHUNK,), jnp.float32),
            pltpu.VMEM((CHUNK,), jnp.float32),
            pltpu.VMEM((CHUNK,), jnp.float32),
            pltpu.SemaphoreType.DMA,
            pltpu.SemaphoreType.DMA,
        ],
    )
    return f(t_tab, u_tab, row_idx, col_idx, px, py, pz)


def _stage3_body(p_ref, d2_ref, wd_ref, b1_ref, we2_ref, b2_ref, wc_ref,
                 bc_ref, m_ref, c_ref):
    pre = p_ref[...] + d2_ref[...] * wd_ref[...] + b1_ref[...]
    h = pre * jax.nn.sigmoid(pre)
    z = jnp.dot(h, we2_ref[...], preferred_element_type=jnp.float32) + b2_ref[...]
    m = z * jax.nn.sigmoid(z)
    m_ref[...] = m
    cz = lax.dot_general(wc_ref[...], m, (((1,), (1,)), ((), ())),
                         preferred_element_type=jnp.float32)
    c_ref[...] = jnp.tanh(cz + bc_ref[...])[0]


def _stage3(p, d2_col, wd_row, b1_row, we2_t, b2_row, wc_row, bc_s):
    blk = 1024
    return pl.pallas_call(
        _stage3_body,
        grid=(E_PAD // blk,),
        in_specs=[
            pl.BlockSpec((blk, FEAT), lambda i: (i, 0)),
            pl.BlockSpec((blk, 1), lambda i: (i, 0)),
            pl.BlockSpec((1, FEAT), lambda i: (0, 0)),
            pl.BlockSpec((1, FEAT), lambda i: (0, 0)),
            pl.BlockSpec((FEAT, FEAT), lambda i: (0, 0)),
            pl.BlockSpec((1, FEAT), lambda i: (0, 0)),
            pl.BlockSpec((1, FEAT), lambda i: (0, 0)),
            pl.BlockSpec((1, 1), lambda i: (0, 0)),
        ],
        out_specs=[
            pl.BlockSpec((blk, FEAT), lambda i: (i, 0)),
            pl.BlockSpec((blk,), lambda i: (i,)),
        ],
        out_shape=[
            jax.ShapeDtypeStruct((E_PAD, FEAT), jnp.float32),
            jax.ShapeDtypeStruct((E_PAD,), jnp.float32),
        ],
    )(p, d2_col, wd_row, b1_row, we2_t, b2_row, wc_row, bc_s)


def _scatter_body(m_hbm, row_hbm, c_hbm, pdx_hbm, pdy_hbm, pdz_hbm,
                  out_hbm, cp_hbm,
                  acc, buf, idx, c_b, pdx_b, pdy_b, pdz_b,
                  accx, accy, accz, sem):
    c = lax.axis_index("c")
    s = lax.axis_index("s")
    wid = c * 16 + s

    def zrow(i, carry2):
        for j in range(FEAT // L):
            buf[i, pl.ds(j * L, L)] = jnp.zeros((L,), jnp.float32)
        return carry2

    lax.fori_loop(0, CHUNK, zrow, 0, unroll=False)

    def zcopy(k, carry2):
        pltpu.sync_copy(buf, acc.at[pl.ds(s * ROWS_PER_TILE + k * CHUNK, CHUNK)])
        return carry2

    lax.fori_loop(0, ROWS_PER_TILE // CHUNK, zcopy, 0, unroll=False)

    def zacc(k, carry2):
        sl = pl.ds(k * L, L)
        z = jnp.zeros((L,), jnp.float32)
        accx[sl] = z
        accy[sl] = z
        accz[sl] = z
        return carry2

    lax.fori_loop(0, N_PAD // L, zacc, 0, unroll=False)
    plsc.subcore_barrier()

    def chunk(g, carry):
        base = pl.multiple_of(wid * EDGES_PER_WORKER + g * CHUNK, CHUNK)
        pltpu.sync_copy(row_hbm.at[pl.ds(base, CHUNK)], idx)
        pltpu.sync_copy(c_hbm.at[pl.ds(base, CHUNK)], c_b)
        pltpu.sync_copy(pdx_hbm.at[pl.ds(base, CHUNK)], pdx_b)
        pltpu.sync_copy(pdy_hbm.at[pl.ds(base, CHUNK)], pdy_b)
        pltpu.sync_copy(pdz_hbm.at[pl.ds(base, CHUNK)], pdz_b)
        pltpu.async_copy(m_hbm.at[pl.ds(base, CHUNK)], buf, sem).wait()
        pltpu.sync_copy(buf, acc.at[idx], add=True)

        for k in range(CHUNK // L):
            sl = pl.ds(k * L, L)
            iv = idx[sl]
            cv = c_b[sl]
            plsc.addupdate_scatter(accx, [iv], pdx_b[sl] * cv)
            plsc.addupdate_scatter(accy, [iv], pdy_b[sl] * cv)
            plsc.addupdate_scatter(accz, [iv], pdz_b[sl] * cv)
        return carry

    lax.fori_loop(0, CHUNKS_PER_WORKER, chunk, 0, unroll=False)
    plsc.subcore_barrier()

    pltpu.sync_copy(acc.at[pl.ds(s * ROWS_PER_TILE, ROWS_PER_TILE)],
                    out_hbm.at[c, pl.ds(s * ROWS_PER_TILE, ROWS_PER_TILE)])
    pltpu.sync_copy(accx, cp_hbm.at[0, wid])
    pltpu.sync_copy(accy, cp_hbm.at[1, wid])
    pltpu.sync_copy(accz, cp_hbm.at[2, wid])


def _stage4(m_rows, row_idx, c1d, pdx, pdy, pdz):
    mesh = plsc.VectorSubcoreMesh(core_axis_name="c", subcore_axis_name="s")
    f = pl.kernel(
        _scatter_body,
        compiler_params=pltpu.CompilerParams(needs_layout_passes=False),
        out_type=[
            jax.ShapeDtypeStruct((2, N_PAD, FEAT), jnp.float32),
            jax.ShapeDtypeStruct((3, NW, N_PAD), jnp.float32),
        ],
        mesh=mesh,
        scratch_types=[
            pltpu.VMEM_SHARED((N_PAD, FEAT), jnp.float32),
            pltpu.VMEM((CHUNK, FEAT), jnp.float32),
            pltpu.VMEM((CHUNK,), jnp.int32),
            pltpu.VMEM((CHUNK,), jnp.float32),
            pltpu.VMEM((CHUNK,), jnp.float32),
            pltpu.VMEM((CHUNK,), jnp.float32),
            pltpu.VMEM((CHUNK,), jnp.float32),
            pltpu.VMEM((N_PAD,), jnp.float32),
            pltpu.VMEM((N_PAD,), jnp.float32),
            pltpu.VMEM((N_PAD,), jnp.float32),
            pltpu.SemaphoreType.DMA,
        ],
    )
    return f(m_rows, row_idx, c1d, pdx, pdy, pdz)


def _stage5_body(a_ref, cp_ref, xc_ref, post_ref, wna_ref, bn_ref,
                 xo_ref, pot_ref):
    agg = a_ref[0] + a_ref[1]
    z = (xc_ref[...] + jnp.dot(agg, wna_ref[...], preferred_element_type=jnp.float32)
         + bn_ref[...])
    xo_ref[...] = z * jax.nn.sigmoid(z)
    pot_ref[...] = post_ref[...] + jnp.sum(cp_ref[...], axis=1)


def _stage5(acc2, cp, xc, pos_t, wna_t, bn_row):
    blk = 1024
    return pl.pallas_call(
        _stage5_body,
        grid=(N_PAD // blk,),
        in_specs=[
            pl.BlockSpec((2, blk, FEAT), lambda i: (0, i, 0)),
            pl.BlockSpec((3, NW, blk), lambda i: (0, 0, i)),
            pl.BlockSpec((blk, FEAT), lambda i: (i, 0)),
            pl.BlockSpec((3, blk), lambda i: (0, i)),
            pl.BlockSpec((FEAT, FEAT), lambda i: (0, 0)),
            pl.BlockSpec((1, FEAT), lambda i: (0, 0)),
        ],
        out_specs=[
            pl.BlockSpec((blk, FEAT), lambda i: (i, 0)),
            pl.BlockSpec((3, blk), lambda i: (0, i)),
        ],
        out_shape=[
            jax.ShapeDtypeStruct((N_PAD, FEAT), jnp.float32),
            jax.ShapeDtypeStruct((3, N_PAD), jnp.float32),
        ],
    )(acc2, cp, xc, pos_t, wna_t, bn_row)


def kernel(x, pos, edge_index, W_e1, b_e1, W_e2, b_e2, W_n, b_n, W_c, b_c):
    row = edge_index[0].astype(jnp.int32)
    col = edge_index[1].astype(jnp.int32)
    row_pad = jnp.concatenate(
        [row, jnp.full((E_PAD - N_EDGES,), N_NODES, jnp.int32)])
    col_pad = jnp.concatenate(
        [col, jnp.zeros((E_PAD - N_EDGES,), jnp.int32)])

    x_pad = jnp.pad(x, ((0, N_PAD - N_NODES), (0, 0)))
    pos_t = jnp.pad(pos, ((0, N_PAD - N_NODES), (0, 0))).T
    px, py, pz = pos_t[0], pos_t[1], pos_t[2]

    wa_t = W_e1[:, :FEAT].T
    wb_t = W_e1[:, FEAT:2 * FEAT].T
    wd_row = W_e1[:, 2 * FEAT].reshape(1, HID)
    wnx_t = W_n[:, :FEAT].T
    wna_t = W_n[:, FEAT:].T

    t_tab, u_tab, xc = _stage1(x_pad, wa_t, wb_t, wnx_t)
    p, pdx, pdy, pdz, d2 = _stage2(t_tab, u_tab, row_pad, col_pad, px, py, pz)
    m_rows, c1d = _stage3(p, d2.reshape(E_PAD, 1), wd_row,
                          b_e1.reshape(1, HID), W_e2.T, b_e2.reshape(1, HID),
                          W_c.reshape(1, HID), b_c.reshape(1, 1))
    acc2, cp = _stage4(m_rows, row_pad, c1d, pdx, pdy, pdz)
    x_out_pad, pos_out_t = _stage5(acc2, cp, xc, pos_t, wna_t,
                                   b_n.reshape(1, FEAT))

    return (x_out_pad[:N_NODES], pos_out_t[:, :N_NODES].T)


# speedup vs baseline: 4.0515x; 1.3472x over previous
import jax
import jax.numpy as jnp
from jax import lax
from jax.experimental import pallas as pl
from jax.experimental.pallas import tpu as pltpu
from jax.experimental.pallas import tpu_sc as plsc

N_NODES = 10000
N_EDGES = 320000
FEAT = 128
HID = 128
N_PAD = 10240
E_PAD = 327680
NW = 32
CHUNK = 128
EDGES_PER_WORKER = E_PAD // NW
CHUNKS_PER_WORKER = EDGES_PER_WORKER // CHUNK
ROWS_PER_TILE = N_PAD // 16
L = 16


def _stage1_body(x_ref, wa_ref, wb_ref, wnx_ref, t_ref, u_ref, xc_ref):
    x = x_ref[...]
    t_ref[...] = jnp.dot(x, wa_ref[...], preferred_element_type=jnp.float32)
    u_ref[...] = jnp.dot(x, wb_ref[...], preferred_element_type=jnp.float32)
    xc_ref[...] = jnp.dot(x, wnx_ref[...], preferred_element_type=jnp.float32)


def _stage1(x_pad, wa_t, wb_t, wnx_t):
    blk = 1024
    return pl.pallas_call(
        _stage1_body,
        grid=(N_PAD // blk,),
        in_specs=[
            pl.BlockSpec((blk, FEAT), lambda i: (i, 0)),
            pl.BlockSpec((FEAT, FEAT), lambda i: (0, 0)),
            pl.BlockSpec((FEAT, FEAT), lambda i: (0, 0)),
            pl.BlockSpec((FEAT, FEAT), lambda i: (0, 0)),
        ],
        out_specs=[
            pl.BlockSpec((blk, FEAT), lambda i: (i, 0)),
            pl.BlockSpec((blk, FEAT), lambda i: (i, 0)),
            pl.BlockSpec((blk, FEAT), lambda i: (i, 0)),
        ],
        out_shape=[
            jax.ShapeDtypeStruct((N_PAD, FEAT), jnp.float32),
            jax.ShapeDtypeStruct((N_PAD, FEAT), jnp.float32),
            jax.ShapeDtypeStruct((N_PAD, FEAT), jnp.float32),
        ],
    )(x_pad, wa_t, wb_t, wnx_t)


def _gather_body(t_hbm, u_hbm, row_hbm, col_hbm, px_hbm, py_hbm, pz_hbm,
                 p_hbm, pd4_hbm,
                 idx_r0, idx_r1, idx_c0, idx_c1, bt0, bt1, bu0, bu1,
                 posx, posy, posz, pd0, pd1,
                 sem_idx, sem_t, sem_u, semo0, semo1):
    c = lax.axis_index("c")
    s = lax.axis_index("s")
    wid = c * 16 + s
    wbase = wid * EDGES_PER_WORKER
    idx_r = [idx_r0, idx_r1]
    idx_c = [idx_c0, idx_c1]
    bt = [bt0, bt1]
    bu = [bu0, bu1]
    pd = [pd0, pd1]
    semo = [semo0, semo1]

    pltpu.sync_copy(px_hbm, posx)
    pltpu.sync_copy(py_hbm, posy)
    pltpu.sync_copy(pz_hbm, posz)

    def front(g, b):
        base = pl.multiple_of(wbase + g * CHUNK, CHUNK)
        pltpu.make_async_copy(row_hbm.at[pl.ds(base, CHUNK)], idx_r[b],
                              sem_idx).wait()
        pltpu.make_async_copy(col_hbm.at[pl.ds(base, CHUNK)], idx_c[b],
                              sem_idx).wait()
        pltpu.async_copy(t_hbm.at[idx_r[b]], bt[b], sem_t)
        pltpu.async_copy(u_hbm.at[idx_c[b]], bu[b], sem_u)
        for k in range(CHUNK // L):
            sl = pl.ds(k * L, L)
            ir = idx_r[b][sl]
            ic = idx_c[b][sl]
            dx = plsc.load_gather(posx, [ir]) - plsc.load_gather(posx, [ic])
            dy = plsc.load_gather(posy, [ir]) - plsc.load_gather(posy, [ic])
            dz = plsc.load_gather(posz, [ir]) - plsc.load_gather(posz, [ic])
            pd[b][0, sl] = dx
            pd[b][1, sl] = dy
            pd[b][2, sl] = dz
            pd[b][3, sl] = dx * dx + dy * dy + dz * dz
        pltpu.async_copy(pd[b], pd4_hbm.at[wid, g], semo[b])

    def back(g, b):
        base = pl.multiple_of(wbase + g * CHUNK, CHUNK)
        pltpu.make_async_copy(t_hbm.at[idx_r[b]], bt[b], sem_t).wait()
        pltpu.make_async_copy(u_hbm.at[idx_c[b]], bu[b], sem_u).wait()

        def add_row(i, carry2):
            for j in range(FEAT // L):
                sl2 = pl.ds(j * L, L)
                plsc.addupdate(bt[b].at[i, sl2], bu[b][i, sl2])
            return carry2

        lax.fori_loop(0, CHUNK, add_row, 0, unroll=False)
        pltpu.async_copy(bt[b], p_hbm.at[pl.ds(base, CHUNK)], semo[b])

    def issue_idx(g, b):
        base = pl.multiple_of(wbase + g * CHUNK, CHUNK)
        pltpu.async_copy(row_hbm.at[pl.ds(base, CHUNK)], idx_r[b], sem_idx)
        pltpu.async_copy(col_hbm.at[pl.ds(base, CHUNK)], idx_c[b], sem_idx)

    def drain_out(g, b):
        base = pl.multiple_of(wbase + g * CHUNK, CHUNK)
        pltpu.make_async_copy(pd[b], pd4_hbm.at[wid, g], semo[b]).wait()
        pltpu.make_async_copy(bt[b], p_hbm.at[pl.ds(base, CHUNK)],
                              semo[b]).wait()

    issue_idx(0, 0)
    issue_idx(1, 1)

    def pair(i, carry):
        for b in range(2):
            g = 2 * i + b

            @pl.when(g >= 2)
            def _():
                drain_out(g - 2, b)

            front(g, b)

            @pl.when(g >= 1)
            def _():
                back(g - 1, 1 - b)

            @pl.when(jnp.logical_and(g >= 1, g + 1 < CHUNKS_PER_WORKER))
            def _():
                issue_idx(g + 1, 1 - b)
        return carry

    lax.fori_loop(0, CHUNKS_PER_WORKER // 2, pair, 0, unroll=False)

    back(CHUNKS_PER_WORKER - 1, (CHUNKS_PER_WORKER - 1) % 2)
    drain_out(CHUNKS_PER_WORKER - 2, (CHUNKS_PER_WORKER - 2) % 2)
    drain_out(CHUNKS_PER_WORKER - 1, (CHUNKS_PER_WORKER - 1) % 2)


def _stage2(t_tab, u_tab, row_idx, col_idx, px, py, pz):
    mesh = plsc.VectorSubcoreMesh(core_axis_name="c", subcore_axis_name="s")
    f = pl.kernel(
        _gather_body,
        compiler_params=pltpu.CompilerParams(needs_layout_passes=False),
        out_type=[
            jax.ShapeDtypeStruct((E_PAD, FEAT), jnp.float32),
            jax.ShapeDtypeStruct((NW, CHUNKS_PER_WORKER, 4, CHUNK),
                                 jnp.float32),
        ],
        mesh=mesh,
        scratch_types=[
            pltpu.VMEM((CHUNK,), jnp.int32),
            pltpu.VMEM((CHUNK,), jnp.int32),
            pltpu.VMEM((CHUNK,), jnp.int32),
            pltpu.VMEM((CHUNK,), jnp.int32),
            pltpu.VMEM((CHUNK, FEAT), jnp.float32),
            pltpu.VMEM((CHUNK, FEAT), jnp.float32),
            pltpu.VMEM((CHUNK, FEAT), jnp.float32),
            pltpu.VMEM((CHUNK, FEAT), jnp.float32),
            pltpu.VMEM((N_PAD,), jnp.float32),
            pltpu.VMEM((N_PAD,), jnp.float32),
            pltpu.VMEM((N_PAD,), jnp.float32),
            pltpu.VMEM((4, CHUNK), jnp.float32),
            pltpu.VMEM((4, CHUNK), jnp.float32),
            pltpu.SemaphoreType.DMA,
            pltpu.SemaphoreType.DMA,
            pltpu.SemaphoreType.DMA,
            pltpu.SemaphoreType.DMA,
            pltpu.SemaphoreType.DMA,
        ],
    )
    return f(t_tab, u_tab, row_idx, col_idx, px, py, pz)


def _stage3_body(p_ref, d2_ref, wd_ref, b1_ref, we2_ref, b2_ref, wc_ref,
                 bc_ref, m_ref, c_ref):
    pre = p_ref[...] + d2_ref[...] * wd_ref[...] + b1_ref[...]
    h = pre * jax.nn.sigmoid(pre)
    z = jnp.dot(h, we2_ref[...], preferred_element_type=jnp.float32) + b2_ref[...]
    m = z * jax.nn.sigmoid(z)
    m_ref[...] = m
    cz = lax.dot_general(wc_ref[...], m, (((1,), (1,)), ((), ())),
                         preferred_element_type=jnp.float32)
    c_ref[...] = jnp.tanh(cz + bc_ref[...])[0]


def _stage3(p, d2_col, wd_row, b1_row, we2_t, b2_row, wc_row, bc_s):
    blk = 1024
    return pl.pallas_call(
        _stage3_body,
        grid=(E_PAD // blk,),
        in_specs=[
            pl.BlockSpec((blk, FEAT), lambda i: (i, 0)),
            pl.BlockSpec((blk, 1), lambda i: (i, 0)),
            pl.BlockSpec((1, FEAT), lambda i: (0, 0)),
            pl.BlockSpec((1, FEAT), lambda i: (0, 0)),
            pl.BlockSpec((FEAT, FEAT), lambda i: (0, 0)),
            pl.BlockSpec((1, FEAT), lambda i: (0, 0)),
            pl.BlockSpec((1, FEAT), lambda i: (0, 0)),
            pl.BlockSpec((1, 1), lambda i: (0, 0)),
        ],
        out_specs=[
            pl.BlockSpec((blk, FEAT), lambda i: (i, 0)),
            pl.BlockSpec((blk,), lambda i: (i,)),
        ],
        out_shape=[
            jax.ShapeDtypeStruct((E_PAD, FEAT), jnp.float32),
            jax.ShapeDtypeStruct((E_PAD,), jnp.float32),
        ],
    )(p, d2_col, wd_row, b1_row, we2_t, b2_row, wc_row, bc_s)


def _scatter_body(m_hbm, row_hbm, out_hbm,
                  acc, m0, m1, idx0, idx1, semin0, semin1, semsc0, semsc1):
    c = lax.axis_index("c")
    s = lax.axis_index("s")
    wid = c * 16 + s
    wbase = wid * EDGES_PER_WORKER
    mb = [m0, m1]
    idx = [idx0, idx1]
    semin = [semin0, semin1]
    semsc = [semsc0, semsc1]

    def zrow(i, carry2):
        for j in range(FEAT // L):
            m0[i, pl.ds(j * L, L)] = jnp.zeros((L,), jnp.float32)
        return carry2

    lax.fori_loop(0, CHUNK, zrow, 0, unroll=False)

    def zcopy(k, carry2):
        pltpu.sync_copy(m0, acc.at[pl.ds(s * ROWS_PER_TILE + k * CHUNK, CHUNK)])
        return carry2

    lax.fori_loop(0, ROWS_PER_TILE // CHUNK, zcopy, 0, unroll=False)
    plsc.subcore_barrier()

    def issue_in(g, b):
        base = pl.multiple_of(wbase + g * CHUNK, CHUNK)
        pltpu.async_copy(row_hbm.at[pl.ds(base, CHUNK)], idx[b], semin[b])
        pltpu.async_copy(m_hbm.at[pl.ds(base, CHUNK)], mb[b], semin[b])

    def wait_in(g, b):
        base = pl.multiple_of(wbase + g * CHUNK, CHUNK)
        pltpu.make_async_copy(row_hbm.at[pl.ds(base, CHUNK)], idx[b],
                              semin[b]).wait()
        pltpu.make_async_copy(m_hbm.at[pl.ds(base, CHUNK)], mb[b],
                              semin[b]).wait()

    issue_in(0, 0)

    def pair(i, carry):
        for b in range(2):
            g = 2 * i + b
            wait_in(g, b)

            @pl.when(g >= 1)
            def _():
                pltpu.make_async_copy(mb[1 - b], acc.at[idx[1 - b]],
                                      semsc[1 - b]).wait()

            @pl.when(g + 1 < CHUNKS_PER_WORKER)
            def _():
                issue_in(g + 1, 1 - b)

            pltpu.async_copy(mb[b], acc.at[idx[b]], semsc[b], add=True)
        return carry

    lax.fori_loop(0, CHUNKS_PER_WORKER // 2, pair, 0, unroll=False)
    lastb = (CHUNKS_PER_WORKER - 1) % 2
    pltpu.make_async_copy(mb[lastb], acc.at[idx[lastb]], semsc[lastb]).wait()
    plsc.subcore_barrier()

    pltpu.sync_copy(acc.at[pl.ds(s * ROWS_PER_TILE, ROWS_PER_TILE)],
                    out_hbm.at[c, pl.ds(s * ROWS_PER_TILE, ROWS_PER_TILE)])


def _stage4(m_rows, row_idx):
    mesh = plsc.VectorSubcoreMesh(core_axis_name="c", subcore_axis_name="s")
    f = pl.kernel(
        _scatter_body,
        compiler_params=pltpu.CompilerParams(needs_layout_passes=False),
        out_type=jax.ShapeDtypeStruct((2, N_PAD, FEAT), jnp.float32),
        mesh=mesh,
        scratch_types=[
            pltpu.VMEM_SHARED((N_PAD, FEAT), jnp.float32),
            pltpu.VMEM((CHUNK, FEAT), jnp.float32),
            pltpu.VMEM((CHUNK, FEAT), jnp.float32),
            pltpu.VMEM((CHUNK,), jnp.int32),
            pltpu.VMEM((CHUNK,), jnp.int32),
            pltpu.SemaphoreType.DMA,
            pltpu.SemaphoreType.DMA,
            pltpu.SemaphoreType.DMA,
            pltpu.SemaphoreType.DMA,
        ],
    )
    return f(m_rows, row_idx)


def _coord_body(row_hbm, c_hbm, pd4_hbm, cp_hbm,
                idx0, idx1, c0, c1, pd0, pd1,
                accx, accy, accz, semin0, semin1):
    c = lax.axis_index("c")
    s = lax.axis_index("s")
    wid = c * 16 + s
    wbase = wid * EDGES_PER_WORKER
    idx = [idx0, idx1]
    cb = [c0, c1]
    pd = [pd0, pd1]
    semin = [semin0, semin1]

    def zacc(k, carry2):
        sl = pl.ds(k * L, L)
        z = jnp.zeros((L,), jnp.float32)
        accx[sl] = z
        accy[sl] = z
        accz[sl] = z
        return carry2

    lax.fori_loop(0, N_PAD // L, zacc, 0, unroll=False)

    def issue_in(g, b):
        base = pl.multiple_of(wbase + g * CHUNK, CHUNK)
        pltpu.async_copy(row_hbm.at[pl.ds(base, CHUNK)], idx[b], semin[b])
        pltpu.async_copy(c_hbm.at[pl.ds(base, CHUNK)], cb[b], semin[b])
        pltpu.async_copy(pd4_hbm.at[wid, g], pd[b], semin[b])

    def wait_in(g, b):
        base = pl.multiple_of(wbase + g * CHUNK, CHUNK)
        pltpu.make_async_copy(row_hbm.at[pl.ds(base, CHUNK)], idx[b],
                              semin[b]).wait()
        pltpu.make_async_copy(c_hbm.at[pl.ds(base, CHUNK)], cb[b],
                              semin[b]).wait()
        pltpu.make_async_copy(pd4_hbm.at[wid, g], pd[b], semin[b]).wait()

    issue_in(0, 0)

    def pair(i, carry):
        for b in range(2):
            g = 2 * i + b
            wait_in(g, b)

            @pl.when(g + 1 < CHUNKS_PER_WORKER)
            def _():
                issue_in(g + 1, 1 - b)

            for k in range(CHUNK // L):
                sl = pl.ds(k * L, L)
                iv = idx[b][sl]
                cv = cb[b][sl]
                plsc.addupdate_scatter(accx, [iv], pd[b][0, sl] * cv)
                plsc.addupdate_scatter(accy, [iv], pd[b][1, sl] * cv)
                plsc.addupdate_scatter(accz, [iv], pd[b][2, sl] * cv)
        return carry

    lax.fori_loop(0, CHUNKS_PER_WORKER // 2, pair, 0, unroll=False)

    pltpu.sync_copy(accx, cp_hbm.at[0, wid])
    pltpu.sync_copy(accy, cp_hbm.at[1, wid])
    pltpu.sync_copy(accz, cp_hbm.at[2, wid])


def _stage4b(row_idx, c1d, pd4):
    mesh = plsc.VectorSubcoreMesh(core_axis_name="c", subcore_axis_name="s")
    f = pl.kernel(
        _coord_body,
        compiler_params=pltpu.CompilerParams(needs_layout_passes=False),
        out_type=jax.ShapeDtypeStruct((3, NW, N_PAD), jnp.float32),
        mesh=mesh,
        scratch_types=[
            pltpu.VMEM((CHUNK,), jnp.int32),
            pltpu.VMEM((CHUNK,), jnp.int32),
            pltpu.VMEM((CHUNK,), jnp.float32),
            pltpu.VMEM((CHUNK,), jnp.float32),
            pltpu.VMEM((4, CHUNK), jnp.float32),
            pltpu.VMEM((4, CHUNK), jnp.float32),
            pltpu.VMEM((N_PAD,), jnp.float32),
            pltpu.VMEM((N_PAD,), jnp.float32),
            pltpu.VMEM((N_PAD,), jnp.float32),
            pltpu.SemaphoreType.DMA,
            pltpu.SemaphoreType.DMA,
        ],
    )
    return f(row_idx, c1d, pd4)


def _stage5_body(a_ref, cp_ref, xc_ref, post_ref, wna_ref, bn_ref,
                 xo_ref, pot_ref):
    agg = a_ref[0] + a_ref[1]
    z = (xc_ref[...] + jnp.dot(agg, wna_ref[...], preferred_element_type=jnp.float32)
         + bn_ref[...])
    xo_ref[...] = z * jax.nn.sigmoid(z)
    pot_ref[...] = post_ref[...] + jnp.sum(cp_ref[...], axis=1)


def _stage5(acc2, cp, xc, pos_t, wna_t, bn_row):
    blk = 1024
    return pl.pallas_call(
        _stage5_body,
        grid=(N_PAD // blk,),
        in_specs=[
            pl.BlockSpec((2, blk, FEAT), lambda i: (0, i, 0)),
            pl.BlockSpec((3, NW, blk), lambda i: (0, 0, i)),
            pl.BlockSpec((blk, FEAT), lambda i: (i, 0)),
            pl.BlockSpec((3, blk), lambda i: (0, i)),
            pl.BlockSpec((FEAT, FEAT), lambda i: (0, 0)),
            pl.BlockSpec((1, FEAT), lambda i: (0, 0)),
        ],
        out_specs=[
            pl.BlockSpec((blk, FEAT), lambda i: (i, 0)),
            pl.BlockSpec((3, blk), lambda i: (0, i)),
        ],
        out_shape=[
            jax.ShapeDtypeStruct((N_PAD, FEAT), jnp.float32),
            jax.ShapeDtypeStruct((3, N_PAD), jnp.float32),
        ],
    )(acc2, cp, xc, pos_t, wna_t, bn_row)


def kernel(x, pos, edge_index, W_e1, b_e1, W_e2, b_e2, W_n, b_n, W_c, b_c):
    row = edge_index[0].astype(jnp.int32)
    col = edge_index[1].astype(jnp.int32)
    row_pad = jnp.concatenate(
        [row, jnp.full((E_PAD - N_EDGES,), N_NODES, jnp.int32)])
    col_pad = jnp.concatenate(
        [col, jnp.zeros((E_PAD - N_EDGES,), jnp.int32)])

    x_pad = jnp.pad(x, ((0, N_PAD - N_NODES), (0, 0)))
    pos_t = jnp.pad(pos, ((0, N_PAD - N_NODES), (0, 0))).T
    px, py, pz = pos_t[0], pos_t[1], pos_t[2]

    wa_t = W_e1[:, :FEAT].T
    wb_t = W_e1[:, FEAT:2 * FEAT].T
    wd_row = W_e1[:, 2 * FEAT].reshape(1, HID)
    wnx_t = W_n[:, :FEAT].T
    wna_t = W_n[:, FEAT:].T

    t_tab, u_tab, xc = _stage1(x_pad, wa_t, wb_t, wnx_t)
    p, pd4 = _stage2(t_tab, u_tab, row_pad, col_pad, px, py, pz)
    d2 = pd4[:, :, 3, :].reshape(E_PAD, 1)
    m_rows, c1d = _stage3(p, d2, wd_row,
                          b_e1.reshape(1, HID), W_e2.T, b_e2.reshape(1, HID),
                          W_c.reshape(1, HID), b_c.reshape(1, 1))
    acc2 = _stage4(m_rows, row_pad)
    cp = _stage4b(row_pad, c1d, pd4)
    x_out_pad, pos_out_t = _stage5(acc2, cp, xc, pos_t, wna_t,
                                   b_n.reshape(1, FEAT))

    return (x_out_pad[:N_NODES], pos_out_t[:, :N_NODES].T)


# speedup vs baseline: 5.9210x; 1.4614x over previous
import jax
import jax.numpy as jnp
from jax import lax
from jax.experimental import pallas as pl
from jax.experimental.pallas import tpu as pltpu
from jax.experimental.pallas import tpu_sc as plsc

N_NODES = 10000
N_EDGES = 320000
FEAT = 128
HID = 128
N_PAD = 10240
E_PAD = 327680
NW = 32
CHUNK = 128
EDGES_PER_WORKER = E_PAD // NW
CHUNKS_PER_WORKER = EDGES_PER_WORKER // CHUNK
ROWS_PER_TILE = N_PAD // 16
L = 16


def _stage1_body(x_ref, wa_ref, wb_ref, wnx_ref, t_ref, u_ref, xc_ref):
    x = x_ref[...]
    t_ref[...] = jnp.dot(x, wa_ref[...], preferred_element_type=jnp.float32)
    u_ref[...] = jnp.dot(x, wb_ref[...], preferred_element_type=jnp.float32)
    xc_ref[...] = jnp.dot(x, wnx_ref[...], preferred_element_type=jnp.float32)


def _stage1(x_pad, wa_t, wb_t, wnx_t):
    blk = 1024
    return pl.pallas_call(
        _stage1_body,
        grid=(N_PAD // blk,),
        in_specs=[
            pl.BlockSpec((blk, FEAT), lambda i: (i, 0)),
            pl.BlockSpec((FEAT, FEAT), lambda i: (0, 0)),
            pl.BlockSpec((FEAT, FEAT), lambda i: (0, 0)),
            pl.BlockSpec((FEAT, FEAT), lambda i: (0, 0)),
        ],
        out_specs=[
            pl.BlockSpec((blk, FEAT), lambda i: (i, 0)),
            pl.BlockSpec((blk, FEAT), lambda i: (i, 0)),
            pl.BlockSpec((blk, FEAT), lambda i: (i, 0)),
        ],
        out_shape=[
            jax.ShapeDtypeStruct((N_PAD, FEAT), jnp.float32),
            jax.ShapeDtypeStruct((N_PAD, FEAT), jnp.float32),
            jax.ShapeDtypeStruct((N_PAD, FEAT), jnp.float32),
        ],
    )(x_pad, wa_t, wb_t, wnx_t)


def _gather_body(t_hbm, u_hbm, row_hbm, col_hbm, px_hbm, py_hbm, pz_hbm,
                 p_hbm, pd4_hbm,
                 idx_r0, idx_r1, idx_c0, idx_c1, bt0, bt1, bu0, bu1,
                 posx, posy, posz, pd0, pd1,
                 sem_idx, sem_t, sem_u, semo0, semo1):
    c = lax.axis_index("c")
    s = lax.axis_index("s")
    wid = c * 16 + s
    wbase = wid * EDGES_PER_WORKER
    idx_r = [idx_r0, idx_r1]
    idx_c = [idx_c0, idx_c1]
    bt = [bt0, bt1]
    bu = [bu0, bu1]
    pd = [pd0, pd1]
    semo = [semo0, semo1]

    pltpu.sync_copy(px_hbm, posx)
    pltpu.sync_copy(py_hbm, posy)
    pltpu.sync_copy(pz_hbm, posz)

    def front(g, b):
        base = pl.multiple_of(wbase + g * CHUNK, CHUNK)
        pltpu.make_async_copy(row_hbm.at[pl.ds(base, CHUNK)], idx_r[b],
                              sem_idx).wait()
        pltpu.make_async_copy(col_hbm.at[pl.ds(base, CHUNK)], idx_c[b],
                              sem_idx).wait()
        pltpu.async_copy(t_hbm.at[idx_r[b]], bt[b], sem_t)
        pltpu.async_copy(u_hbm.at[idx_c[b]], bu[b], sem_u)
        for k in range(CHUNK // L):
            sl = pl.ds(k * L, L)
            ir = idx_r[b][sl]
            ic = idx_c[b][sl]
            dx = plsc.load_gather(posx, [ir]) - plsc.load_gather(posx, [ic])
            dy = plsc.load_gather(posy, [ir]) - plsc.load_gather(posy, [ic])
            dz = plsc.load_gather(posz, [ir]) - plsc.load_gather(posz, [ic])
            pd[b][0, sl] = dx
            pd[b][1, sl] = dy
            pd[b][2, sl] = dz
            pd[b][3, sl] = dx * dx + dy * dy + dz * dz
        pltpu.async_copy(pd[b], pd4_hbm.at[wid, g], semo[b])

    def back(g, b):
        base = pl.multiple_of(wbase + g * CHUNK, CHUNK)
        pltpu.make_async_copy(t_hbm.at[idx_r[b]], bt[b], sem_t).wait()
        pltpu.make_async_copy(u_hbm.at[idx_c[b]], bu[b], sem_u).wait()

        def add_row(i, carry2):
            for j in range(FEAT // L):
                sl2 = pl.ds(j * L, L)
                plsc.addupdate(bt[b].at[i, sl2], bu[b][i, sl2])
            return carry2

        lax.fori_loop(0, CHUNK, add_row, 0, unroll=False)
        pltpu.async_copy(bt[b], p_hbm.at[pl.ds(base, CHUNK)], semo[b])

    def issue_idx(g, b):
        base = pl.multiple_of(wbase + g * CHUNK, CHUNK)
        pltpu.async_copy(row_hbm.at[pl.ds(base, CHUNK)], idx_r[b], sem_idx)
        pltpu.async_copy(col_hbm.at[pl.ds(base, CHUNK)], idx_c[b], sem_idx)

    def drain_out(g, b):
        base = pl.multiple_of(wbase + g * CHUNK, CHUNK)
        pltpu.make_async_copy(pd[b], pd4_hbm.at[wid, g], semo[b]).wait()
        pltpu.make_async_copy(bt[b], p_hbm.at[pl.ds(base, CHUNK)],
                              semo[b]).wait()

    issue_idx(0, 0)
    issue_idx(1, 1)

    def pair(i, carry):
        for b in range(2):
            g = 2 * i + b

            @pl.when(g >= 2)
            def _():
                drain_out(g - 2, b)

            front(g, b)

            @pl.when(g >= 1)
            def _():
                back(g - 1, 1 - b)

            @pl.when(jnp.logical_and(g >= 1, g + 1 < CHUNKS_PER_WORKER))
            def _():
                issue_idx(g + 1, 1 - b)
        return carry

    lax.fori_loop(0, CHUNKS_PER_WORKER // 2, pair, 0, unroll=False)

    back(CHUNKS_PER_WORKER - 1, (CHUNKS_PER_WORKER - 1) % 2)
    drain_out(CHUNKS_PER_WORKER - 2, (CHUNKS_PER_WORKER - 2) % 2)
    drain_out(CHUNKS_PER_WORKER - 1, (CHUNKS_PER_WORKER - 1) % 2)


def _stage2(t_tab, u_tab, row_idx, col_idx, px, py, pz):
    mesh = plsc.VectorSubcoreMesh(core_axis_name="c", subcore_axis_name="s")
    f = pl.kernel(
        _gather_body,
        compiler_params=pltpu.CompilerParams(needs_layout_passes=False),
        out_type=[
            jax.ShapeDtypeStruct((E_PAD, FEAT), jnp.float32),
            jax.ShapeDtypeStruct((NW, CHUNKS_PER_WORKER, 4, CHUNK),
                                 jnp.float32),
        ],
        mesh=mesh,
        scratch_types=[
            pltpu.VMEM((CHUNK,), jnp.int32),
            pltpu.VMEM((CHUNK,), jnp.int32),
            pltpu.VMEM((CHUNK,), jnp.int32),
            pltpu.VMEM((CHUNK,), jnp.int32),
            pltpu.VMEM((CHUNK, FEAT), jnp.float32),
            pltpu.VMEM((CHUNK, FEAT), jnp.float32),
            pltpu.VMEM((CHUNK, FEAT), jnp.float32),
            pltpu.VMEM((CHUNK, FEAT), jnp.float32),
            pltpu.VMEM((N_PAD,), jnp.float32),
            pltpu.VMEM((N_PAD,), jnp.float32),
            pltpu.VMEM((N_PAD,), jnp.float32),
            pltpu.VMEM((4, CHUNK), jnp.float32),
            pltpu.VMEM((4, CHUNK), jnp.float32),
            pltpu.SemaphoreType.DMA,
            pltpu.SemaphoreType.DMA,
            pltpu.SemaphoreType.DMA,
            pltpu.SemaphoreType.DMA,
            pltpu.SemaphoreType.DMA,
        ],
    )
    return f(t_tab, u_tab, row_idx, col_idx, px, py, pz)


def _stage3_body(p_ref, d2_ref, wd_ref, b1_ref, we2_ref, b2_ref, wc_ref,
                 bc_ref, m_ref, c_ref):
    pre = p_ref[...] + d2_ref[...] * wd_ref[...] + b1_ref[...]
    h = pre * jax.nn.sigmoid(pre)
    z = jnp.dot(h, we2_ref[...], preferred_element_type=jnp.float32) + b2_ref[...]
    m = z * jax.nn.sigmoid(z)
    m_ref[...] = m
    cz = lax.dot_general(wc_ref[...], m, (((1,), (1,)), ((), ())),
                         preferred_element_type=jnp.float32)
    c_ref[...] = jnp.tanh(cz + bc_ref[...])[0]


def _stage3(p, d2_col, wd_row, b1_row, we2_t, b2_row, wc_row, bc_s):
    blk = 1024
    return pl.pallas_call(
        _stage3_body,
        grid=(E_PAD // blk,),
        in_specs=[
            pl.BlockSpec((blk, FEAT), lambda i: (i, 0)),
            pl.BlockSpec((blk, 1), lambda i: (i, 0)),
            pl.BlockSpec((1, FEAT), lambda i: (0, 0)),
            pl.BlockSpec((1, FEAT), lambda i: (0, 0)),
            pl.BlockSpec((FEAT, FEAT), lambda i: (0, 0)),
            pl.BlockSpec((1, FEAT), lambda i: (0, 0)),
            pl.BlockSpec((1, FEAT), lambda i: (0, 0)),
            pl.BlockSpec((1, 1), lambda i: (0, 0)),
        ],
        out_specs=[
            pl.BlockSpec((blk, FEAT), lambda i: (i, 0)),
            pl.BlockSpec((blk,), lambda i: (i,)),
        ],
        out_shape=[
            jax.ShapeDtypeStruct((E_PAD, FEAT), jnp.float32),
            jax.ShapeDtypeStruct((E_PAD,), jnp.float32),
        ],
    )(p, d2_col, wd_row, b1_row, we2_t, b2_row, wc_row, bc_s)


def _scatter_body(m_hbm, row_hbm, out_hbm,
                  acc, m0, m1, idx0, idx1, semin0, semin1, semsc0, semsc1):
    c = lax.axis_index("c")
    s = lax.axis_index("s")
    wid = c * 16 + s
    wbase = wid * EDGES_PER_WORKER
    mb = [m0, m1]
    idx = [idx0, idx1]
    semin = [semin0, semin1]
    semsc = [semsc0, semsc1]

    def zrow(i, carry2):
        for j in range(FEAT // L):
            m0[i, pl.ds(j * L, L)] = jnp.zeros((L,), jnp.float32)
        return carry2

    lax.fori_loop(0, CHUNK, zrow, 0, unroll=False)

    def zcopy(k, carry2):
        pltpu.sync_copy(m0, acc.at[pl.ds(s * ROWS_PER_TILE + k * CHUNK, CHUNK)])
        return carry2

    lax.fori_loop(0, ROWS_PER_TILE // CHUNK, zcopy, 0, unroll=False)
    plsc.subcore_barrier()

    def issue_in(g, b):
        base = pl.multiple_of(wbase + g * CHUNK, CHUNK)
        pltpu.async_copy(row_hbm.at[pl.ds(base, CHUNK)], idx[b], semin[b])
        pltpu.async_copy(m_hbm.at[pl.ds(base, CHUNK)], mb[b], semin[b])

    def wait_in(g, b):
        base = pl.multiple_of(wbase + g * CHUNK, CHUNK)
        pltpu.make_async_copy(row_hbm.at[pl.ds(base, CHUNK)], idx[b],
                              semin[b]).wait()
        pltpu.make_async_copy(m_hbm.at[pl.ds(base, CHUNK)], mb[b],
                              semin[b]).wait()

    issue_in(0, 0)

    def pair(i, carry):
        for b in range(2):
            g = 2 * i + b
            wait_in(g, b)

            @pl.when(g >= 1)
            def _():
                pltpu.make_async_copy(mb[1 - b], acc.at[idx[1 - b]],
                                      semsc[1 - b]).wait()

            @pl.when(g + 1 < CHUNKS_PER_WORKER)
            def _():
                issue_in(g + 1, 1 - b)

            pltpu.async_copy(mb[b], acc.at[idx[b]], semsc[b], add=True)
        return carry

    lax.fori_loop(0, CHUNKS_PER_WORKER // 2, pair, 0, unroll=False)
    lastb = (CHUNKS_PER_WORKER - 1) % 2
    pltpu.make_async_copy(mb[lastb], acc.at[idx[lastb]], semsc[lastb]).wait()
    plsc.subcore_barrier()

    pltpu.sync_copy(acc.at[pl.ds(s * ROWS_PER_TILE, ROWS_PER_TILE)],
                    out_hbm.at[c, pl.ds(s * ROWS_PER_TILE, ROWS_PER_TILE)])


def _stage4(m_rows, row_idx):
    mesh = plsc.VectorSubcoreMesh(core_axis_name="c", subcore_axis_name="s")
    f = pl.kernel(
        _scatter_body,
        compiler_params=pltpu.CompilerParams(needs_layout_passes=False),
        out_type=jax.ShapeDtypeStruct((2, N_PAD, FEAT), jnp.float32),
        mesh=mesh,
        scratch_types=[
            pltpu.VMEM_SHARED((N_PAD, FEAT), jnp.float32),
            pltpu.VMEM((CHUNK, FEAT), jnp.float32),
            pltpu.VMEM((CHUNK, FEAT), jnp.float32),
            pltpu.VMEM((CHUNK,), jnp.int32),
            pltpu.VMEM((CHUNK,), jnp.int32),
            pltpu.SemaphoreType.DMA,
            pltpu.SemaphoreType.DMA,
            pltpu.SemaphoreType.DMA,
            pltpu.SemaphoreType.DMA,
        ],
    )
    return f(m_rows, row_idx)


def _coord_body(row_hbm, c_hbm, pd4_hbm, cp_hbm,
                idx0, idx1, c0, c1, pd0, pd1,
                accx, accy, accz, semin0, semin1):
    c = lax.axis_index("c")
    s = lax.axis_index("s")
    wid = c * 16 + s
    wbase = wid * EDGES_PER_WORKER
    idx = [idx0, idx1]
    cb = [c0, c1]
    pd = [pd0, pd1]
    semin = [semin0, semin1]

    def zacc(k, carry2):
        sl = pl.ds(k * L, L)
        z = jnp.zeros((L,), jnp.float32)
        accx[sl] = z
        accy[sl] = z
        accz[sl] = z
        return carry2

    lax.fori_loop(0, N_PAD // L, zacc, 0, unroll=False)

    def issue_in(g, b):
        base = pl.multiple_of(wbase + g * CHUNK, CHUNK)
        pltpu.async_copy(row_hbm.at[pl.ds(base, CHUNK)], idx[b], semin[b])
        pltpu.async_copy(c_hbm.at[pl.ds(base, CHUNK)], cb[b], semin[b])
        pltpu.async_copy(pd4_hbm.at[wid, g], pd[b], semin[b])

    def wait_in(g, b):
        base = pl.multiple_of(wbase + g * CHUNK, CHUNK)
        pltpu.make_async_copy(row_hbm.at[pl.ds(base, CHUNK)], idx[b],
                              semin[b]).wait()
        pltpu.make_async_copy(c_hbm.at[pl.ds(base, CHUNK)], cb[b],
                              semin[b]).wait()
        pltpu.make_async_copy(pd4_hbm.at[wid, g], pd[b], semin[b]).wait()

    issue_in(0, 0)

    def pair(i, carry):
        for b in range(2):
            g = 2 * i + b
            wait_in(g, b)

            @pl.when(g + 1 < CHUNKS_PER_WORKER)
            def _():
                issue_in(g + 1, 1 - b)

            for k in range(CHUNK // L):
                sl = pl.ds(k * L, L)
                iv = idx[b][sl]
                cv = cb[b][sl]
                plsc.addupdate_scatter(accx, [iv], pd[b][0, sl] * cv)
                plsc.addupdate_scatter(accy, [iv], pd[b][1, sl] * cv)
                plsc.addupdate_scatter(accz, [iv], pd[b][2, sl] * cv)
        return carry

    lax.fori_loop(0, CHUNKS_PER_WORKER // 2, pair, 0, unroll=False)

    pltpu.sync_copy(accx, cp_hbm.at[0, wid])
    pltpu.sync_copy(accy, cp_hbm.at[1, wid])
    pltpu.sync_copy(accz, cp_hbm.at[2, wid])


def _stage4b(row_idx, c1d, pd4):
    mesh = plsc.VectorSubcoreMesh(core_axis_name="c", subcore_axis_name="s")
    f = pl.kernel(
        _coord_body,
        compiler_params=pltpu.CompilerParams(needs_layout_passes=False),
        out_type=jax.ShapeDtypeStruct((3, NW, N_PAD), jnp.float32),
        mesh=mesh,
        scratch_types=[
            pltpu.VMEM((CHUNK,), jnp.int32),
            pltpu.VMEM((CHUNK,), jnp.int32),
            pltpu.VMEM((CHUNK,), jnp.float32),
            pltpu.VMEM((CHUNK,), jnp.float32),
            pltpu.VMEM((4, CHUNK), jnp.float32),
            pltpu.VMEM((4, CHUNK), jnp.float32),
            pltpu.VMEM((N_PAD,), jnp.float32),
            pltpu.VMEM((N_PAD,), jnp.float32),
            pltpu.VMEM((N_PAD,), jnp.float32),
            pltpu.SemaphoreType.DMA,
            pltpu.SemaphoreType.DMA,
        ],
    )
    return f(row_idx, c1d, pd4)


def _stage5_body(a_ref, cp_ref, xc_ref, post_ref, wna_ref, bn_ref,
                 xo_ref, pot_ref):
    agg = a_ref[0] + a_ref[1]
    z = (xc_ref[...] + jnp.dot(agg, wna_ref[...], preferred_element_type=jnp.float32)
         + bn_ref[...])
    xo_ref[...] = z * jax.nn.sigmoid(z)
    pot_ref[...] = post_ref[...] + jnp.sum(cp_ref[...], axis=1)


def _stage5(acc2, cp, xc, pos_t, wna_t, bn_row):
    blk = 1024
    return pl.pallas_call(
        _stage5_body,
        grid=(N_PAD // blk,),
        in_specs=[
            pl.BlockSpec((2, blk, FEAT), lambda i: (0, i, 0)),
            pl.BlockSpec((3, NW, blk), lambda i: (0, 0, i)),
            pl.BlockSpec((blk, FEAT), lambda i: (i, 0)),
            pl.BlockSpec((3, blk), lambda i: (0, i)),
            pl.BlockSpec((FEAT, FEAT), lambda i: (0, 0)),
            pl.BlockSpec((1, FEAT), lambda i: (0, 0)),
        ],
        out_specs=[
            pl.BlockSpec((blk, FEAT), lambda i: (i, 0)),
            pl.BlockSpec((3, blk), lambda i: (0, i)),
        ],
        out_shape=[
            jax.ShapeDtypeStruct((N_PAD, FEAT), jnp.float32),
            jax.ShapeDtypeStruct((3, N_PAD), jnp.float32),
        ],
    )(acc2, cp, xc, pos_t, wna_t, bn_row)


def kernel(x, pos, edge_index, W_e1, b_e1, W_e2, b_e2, W_n, b_n, W_c, b_c):
    row = edge_index[0].astype(jnp.int32)
    col = edge_index[1].astype(jnp.int32)
    pad_iota = jnp.arange(E_PAD - N_EDGES, dtype=jnp.int32)
    row_pad = jnp.concatenate([row, N_NODES + pad_iota % (N_PAD - N_NODES)])
    col_pad = jnp.concatenate([col, pad_iota % N_PAD])

    x_pad = jnp.pad(x, ((0, N_PAD - N_NODES), (0, 0)))
    pos_t = jnp.pad(pos, ((0, N_PAD - N_NODES), (0, 0))).T
    px, py, pz = pos_t[0], pos_t[1], pos_t[2]

    wa_t = W_e1[:, :FEAT].T
    wb_t = W_e1[:, FEAT:2 * FEAT].T
    wd_row = W_e1[:, 2 * FEAT].reshape(1, HID)
    wnx_t = W_n[:, :FEAT].T
    wna_t = W_n[:, FEAT:].T

    t_tab, u_tab, xc = _stage1(x_pad, wa_t, wb_t, wnx_t)
    p, pd4 = _stage2(t_tab, u_tab, row_pad, col_pad, px, py, pz)
    d2 = pd4[:, :, 3, :].reshape(E_PAD, 1)
    m_rows, c1d = _stage3(p, d2, wd_row,
                          b_e1.reshape(1, HID), W_e2.T, b_e2.reshape(1, HID),
                          W_c.reshape(1, HID), b_c.reshape(1, 1))
    acc2 = _stage4(m_rows, row_pad)
    cp = _stage4b(row_pad, c1d, pd4)
    x_out_pad, pos_out_t = _stage5(acc2, cp, xc, pos_t, wna_t,
                                   b_n.reshape(1, FEAT))

    return (x_out_pad[:N_NODES], pos_out_t[:, :N_NODES].T)


# speedup vs baseline: 6.1275x; 1.0349x over previous
import jax
import jax.numpy as jnp
from jax import lax
from jax.experimental import pallas as pl
from jax.experimental.pallas import tpu as pltpu
from jax.experimental.pallas import tpu_sc as plsc

N_NODES = 10000
N_EDGES = 320000
FEAT = 128
HID = 128
N_PAD = 10240
E_PAD = 327680
NW = 32
CHUNK = 128
EDGES_PER_WORKER = E_PAD // NW
CHUNKS_PER_WORKER = EDGES_PER_WORKER // CHUNK
ROWS_PER_TILE = N_PAD // 16
L = 16


def _stage1_body(x_ref, wa_ref, wb_ref, wnx_ref, t_ref, u_ref, xc_ref):
    x = x_ref[...]
    t_ref[...] = jnp.dot(x, wa_ref[...], preferred_element_type=jnp.float32)
    u_ref[...] = jnp.dot(x, wb_ref[...], preferred_element_type=jnp.float32)
    xc_ref[...] = jnp.dot(x, wnx_ref[...], preferred_element_type=jnp.float32)


def _stage1(x_pad, wa_t, wb_t, wnx_t):
    blk = 1024
    return pl.pallas_call(
        _stage1_body,
        grid=(N_PAD // blk,),
        in_specs=[
            pl.BlockSpec((blk, FEAT), lambda i: (i, 0)),
            pl.BlockSpec((FEAT, FEAT), lambda i: (0, 0)),
            pl.BlockSpec((FEAT, FEAT), lambda i: (0, 0)),
            pl.BlockSpec((FEAT, FEAT), lambda i: (0, 0)),
        ],
        out_specs=[
            pl.BlockSpec((blk, FEAT), lambda i: (i, 0)),
            pl.BlockSpec((blk, FEAT), lambda i: (i, 0)),
            pl.BlockSpec((blk, FEAT), lambda i: (i, 0)),
        ],
        out_shape=[
            jax.ShapeDtypeStruct((N_PAD, FEAT), jnp.float32),
            jax.ShapeDtypeStruct((N_PAD, FEAT), jnp.float32),
            jax.ShapeDtypeStruct((N_PAD, FEAT), jnp.float32),
        ],
    )(x_pad, wa_t, wb_t, wnx_t)


def _gather_body(t_hbm, u_hbm, row_hbm, col_hbm, px_hbm, py_hbm, pz_hbm,
                 p_hbm, pd4_hbm,
                 idx_r0, idx_r1, idx_c0, idx_c1, bt0, bt1, bu0, bu1,
                 posx, posy, posz, pd0, pd1,
                 sem_idx, sem_t, sem_u, semo0, semo1):
    c = lax.axis_index("c")
    s = lax.axis_index("s")
    wid = c * 16 + s
    wbase = wid * EDGES_PER_WORKER
    idx_r = [idx_r0, idx_r1]
    idx_c = [idx_c0, idx_c1]
    bt = [bt0, bt1]
    bu = [bu0, bu1]
    pd = [pd0, pd1]
    semo = [semo0, semo1]

    pltpu.sync_copy(px_hbm, posx)
    pltpu.sync_copy(py_hbm, posy)
    pltpu.sync_copy(pz_hbm, posz)

    def front(g, b):
        base = pl.multiple_of(wbase + g * CHUNK, CHUNK)
        pltpu.make_async_copy(row_hbm.at[pl.ds(base, CHUNK)], idx_r[b],
                              sem_idx).wait()
        pltpu.make_async_copy(col_hbm.at[pl.ds(base, CHUNK)], idx_c[b],
                              sem_idx).wait()
        pltpu.async_copy(t_hbm.at[idx_r[b]], bt[b], sem_t)
        pltpu.async_copy(u_hbm.at[idx_c[b]], bu[b], sem_u)
        for k in range(CHUNK // L):
            sl = pl.ds(k * L, L)
            ir = idx_r[b][sl]
            ic = idx_c[b][sl]
            dx = plsc.load_gather(posx, [ir]) - plsc.load_gather(posx, [ic])
            dy = plsc.load_gather(posy, [ir]) - plsc.load_gather(posy, [ic])
            dz = plsc.load_gather(posz, [ir]) - plsc.load_gather(posz, [ic])
            pd[b][0, sl] = dx
            pd[b][1, sl] = dy
            pd[b][2, sl] = dz
            pd[b][3, sl] = dx * dx + dy * dy + dz * dz
        pltpu.async_copy(pd[b], pd4_hbm.at[wid, g], semo[b])

    def back(g, b):
        base = pl.multiple_of(wbase + g * CHUNK, CHUNK)
        pltpu.make_async_copy(t_hbm.at[idx_r[b]], bt[b], sem_t).wait()
        pltpu.make_async_copy(u_hbm.at[idx_c[b]], bu[b], sem_u).wait()

        def add_row(i, carry2):
            for j in range(FEAT // L):
                sl2 = pl.ds(j * L, L)
                plsc.addupdate(bt[b].at[i, sl2], bu[b][i, sl2])
            return carry2

        lax.fori_loop(0, CHUNK, add_row, 0, unroll=False)
        pltpu.async_copy(bt[b], p_hbm.at[pl.ds(base, CHUNK)], semo[b])

    def issue_idx(g, b):
        base = pl.multiple_of(wbase + g * CHUNK, CHUNK)
        pltpu.async_copy(row_hbm.at[pl.ds(base, CHUNK)], idx_r[b], sem_idx)
        pltpu.async_copy(col_hbm.at[pl.ds(base, CHUNK)], idx_c[b], sem_idx)

    def drain_out(g, b):
        base = pl.multiple_of(wbase + g * CHUNK, CHUNK)
        pltpu.make_async_copy(pd[b], pd4_hbm.at[wid, g], semo[b]).wait()
        pltpu.make_async_copy(bt[b], p_hbm.at[pl.ds(base, CHUNK)],
                              semo[b]).wait()

    issue_idx(0, 0)
    issue_idx(1, 1)

    def pair(i, carry):
        for b in range(2):
            g = 2 * i + b

            @pl.when(g >= 2)
            def _():
                drain_out(g - 2, b)

            front(g, b)

            @pl.when(g >= 1)
            def _():
                back(g - 1, 1 - b)

            @pl.when(jnp.logical_and(g >= 1, g + 1 < CHUNKS_PER_WORKER))
            def _():
                issue_idx(g + 1, 1 - b)
        return carry

    lax.fori_loop(0, CHUNKS_PER_WORKER // 2, pair, 0, unroll=False)

    back(CHUNKS_PER_WORKER - 1, (CHUNKS_PER_WORKER - 1) % 2)
    drain_out(CHUNKS_PER_WORKER - 2, (CHUNKS_PER_WORKER - 2) % 2)
    drain_out(CHUNKS_PER_WORKER - 1, (CHUNKS_PER_WORKER - 1) % 2)


def _stage2(t_tab, u_tab, row_idx, col_idx, px, py, pz):
    mesh = plsc.VectorSubcoreMesh(core_axis_name="c", subcore_axis_name="s")
    f = pl.kernel(
        _gather_body,
        compiler_params=pltpu.CompilerParams(needs_layout_passes=False),
        out_type=[
            jax.ShapeDtypeStruct((E_PAD, FEAT), jnp.float32),
            jax.ShapeDtypeStruct((NW, CHUNKS_PER_WORKER, 4, CHUNK),
                                 jnp.float32),
        ],
        mesh=mesh,
        scratch_types=[
            pltpu.VMEM((CHUNK,), jnp.int32),
            pltpu.VMEM((CHUNK,), jnp.int32),
            pltpu.VMEM((CHUNK,), jnp.int32),
            pltpu.VMEM((CHUNK,), jnp.int32),
            pltpu.VMEM((CHUNK, FEAT), jnp.float32),
            pltpu.VMEM((CHUNK, FEAT), jnp.float32),
            pltpu.VMEM((CHUNK, FEAT), jnp.float32),
            pltpu.VMEM((CHUNK, FEAT), jnp.float32),
            pltpu.VMEM((N_PAD,), jnp.float32),
            pltpu.VMEM((N_PAD,), jnp.float32),
            pltpu.VMEM((N_PAD,), jnp.float32),
            pltpu.VMEM((4, CHUNK), jnp.float32),
            pltpu.VMEM((4, CHUNK), jnp.float32),
            pltpu.SemaphoreType.DMA,
            pltpu.SemaphoreType.DMA,
            pltpu.SemaphoreType.DMA,
            pltpu.SemaphoreType.DMA,
            pltpu.SemaphoreType.DMA,
        ],
    )
    return f(t_tab, u_tab, row_idx, col_idx, px, py, pz)


def _stage3_body(p_ref, pd4_ref, wd_ref, b1_ref, we2_ref, b2_ref, wc_ref,
                 bc_ref, m_ref, c_ref):
    wd = wd_ref[...]
    b1 = b1_ref[...]
    b2 = b2_ref[...]
    we2 = we2_ref[...]
    wc = wc_ref[...]
    bc = bc_ref[...]
    d2t = jnp.transpose(pd4_ref[0, :, 3, :], (1, 0))
    for i in range(8):
        pre = p_ref[i] + d2t[:, i:i + 1] * wd + b1
        h = pre * jax.nn.sigmoid(pre)
        z = jnp.dot(h.astype(jnp.bfloat16), we2,
                    preferred_element_type=jnp.float32) + b2
        m = z * jax.nn.sigmoid(z)
        m_ref[i] = m
        cz = lax.dot_general(wc, m, (((1,), (1,)), ((), ())),
                             preferred_element_type=jnp.float32)
        c_ref[pl.ds(i, 1), :] = jnp.tanh(cz + bc)


def _stage3(p3, pd4, wd_row, b1_row, we2_bf, b2_row, wc_row, bc_s):
    nblk = E_PAD // (8 * CHUNK)
    gpw = CHUNKS_PER_WORKER // 8
    return pl.pallas_call(
        _stage3_body,
        grid=(nblk,),
        in_specs=[
            pl.BlockSpec((8, CHUNK, FEAT), lambda i: (i, 0, 0)),
            pl.BlockSpec((1, 8, 4, CHUNK), lambda i: (i // gpw, i % gpw, 0, 0)),
            pl.BlockSpec((1, FEAT), lambda i: (0, 0)),
            pl.BlockSpec((1, FEAT), lambda i: (0, 0)),
            pl.BlockSpec((FEAT, FEAT), lambda i: (0, 0)),
            pl.BlockSpec((1, FEAT), lambda i: (0, 0)),
            pl.BlockSpec((1, FEAT), lambda i: (0, 0)),
            pl.BlockSpec((1, 1), lambda i: (0, 0)),
        ],
        out_specs=[
            pl.BlockSpec((8, CHUNK, FEAT), lambda i: (i, 0, 0)),
            pl.BlockSpec((8, CHUNK), lambda i: (i, 0)),
        ],
        out_shape=[
            jax.ShapeDtypeStruct((E_PAD // CHUNK, CHUNK, FEAT), jnp.float32),
            jax.ShapeDtypeStruct((E_PAD // CHUNK, CHUNK), jnp.float32),
        ],
    )(p3, pd4, wd_row, b1_row, we2_bf, b2_row, wc_row, bc_s)


def _scatter_body(m_hbm, row_hbm, out_hbm,
                  acc, m0, m1, idx0, idx1, semin0, semin1, semsc0, semsc1):
    c = lax.axis_index("c")
    s = lax.axis_index("s")
    wid = c * 16 + s
    wbase = wid * EDGES_PER_WORKER
    mb = [m0, m1]
    idx = [idx0, idx1]
    semin = [semin0, semin1]
    semsc = [semsc0, semsc1]

    def zrow(i, carry2):
        for j in range(FEAT // L):
            m0[i, pl.ds(j * L, L)] = jnp.zeros((L,), jnp.float32)
        return carry2

    lax.fori_loop(0, CHUNK, zrow, 0, unroll=False)

    def zcopy(k, carry2):
        pltpu.sync_copy(m0, acc.at[pl.ds(s * ROWS_PER_TILE + k * CHUNK, CHUNK)])
        return carry2

    lax.fori_loop(0, ROWS_PER_TILE // CHUNK, zcopy, 0, unroll=False)
    plsc.subcore_barrier()

    def issue_in(g, b):
        base = pl.multiple_of(wbase + g * CHUNK, CHUNK)
        pltpu.async_copy(row_hbm.at[pl.ds(base, CHUNK)], idx[b], semin[b])
        pltpu.async_copy(m_hbm.at[pl.ds(base, CHUNK)], mb[b], semin[b])

    def wait_in(g, b):
        base = pl.multiple_of(wbase + g * CHUNK, CHUNK)
        pltpu.make_async_copy(row_hbm.at[pl.ds(base, CHUNK)], idx[b],
                              semin[b]).wait()
        pltpu.make_async_copy(m_hbm.at[pl.ds(base, CHUNK)], mb[b],
                              semin[b]).wait()

    issue_in(0, 0)

    def pair(i, carry):
        for b in range(2):
            g = 2 * i + b
            wait_in(g, b)

            @pl.when(g >= 1)
            def _():
                pltpu.make_async_copy(mb[1 - b], acc.at[idx[1 - b]],
                                      semsc[1 - b]).wait()

            @pl.when(g + 1 < CHUNKS_PER_WORKER)
            def _():
                issue_in(g + 1, 1 - b)

            pltpu.async_copy(mb[b], acc.at[idx[b]], semsc[b], add=True)
        return carry

    lax.fori_loop(0, CHUNKS_PER_WORKER // 2, pair, 0, unroll=False)
    lastb = (CHUNKS_PER_WORKER - 1) % 2
    pltpu.make_async_copy(mb[lastb], acc.at[idx[lastb]], semsc[lastb]).wait()
    plsc.subcore_barrier()

    pltpu.sync_copy(acc.at[pl.ds(s * ROWS_PER_TILE, ROWS_PER_TILE)],
                    out_hbm.at[c, pl.ds(s * ROWS_PER_TILE, ROWS_PER_TILE)])


def _stage4(m_rows, row_idx):
    mesh = plsc.VectorSubcoreMesh(core_axis_name="c", subcore_axis_name="s")
    f = pl.kernel(
        _scatter_body,
        compiler_params=pltpu.CompilerParams(needs_layout_passes=False),
        out_type=jax.ShapeDtypeStruct((2, N_PAD, FEAT), jnp.float32),
        mesh=mesh,
        scratch_types=[
            pltpu.VMEM_SHARED((N_PAD, FEAT), jnp.float32),
            pltpu.VMEM((CHUNK, FEAT), jnp.float32),
            pltpu.VMEM((CHUNK, FEAT), jnp.float32),
            pltpu.VMEM((CHUNK,), jnp.int32),
            pltpu.VMEM((CHUNK,), jnp.int32),
            pltpu.SemaphoreType.DMA,
            pltpu.SemaphoreType.DMA,
            pltpu.SemaphoreType.DMA,
            pltpu.SemaphoreType.DMA,
        ],
    )
    return f(m_rows, row_idx)


def _coord_body(row_hbm, c_hbm, pd4_hbm, cp_hbm,
                idx0, idx1, c0, c1, pd0, pd1,
                accx, accy, accz, semin0, semin1):
    c = lax.axis_index("c")
    s = lax.axis_index("s")
    wid = c * 16 + s
    wbase = wid * EDGES_PER_WORKER
    idx = [idx0, idx1]
    cb = [c0, c1]
    pd = [pd0, pd1]
    semin = [semin0, semin1]

    def zacc(k, carry2):
        sl = pl.ds(k * L, L)
        z = jnp.zeros((L,), jnp.float32)
        accx[sl] = z
        accy[sl] = z
        accz[sl] = z
        return carry2

    lax.fori_loop(0, N_PAD // L, zacc, 0, unroll=False)

    def issue_in(g, b):
        base = pl.multiple_of(wbase + g * CHUNK, CHUNK)
        pltpu.async_copy(row_hbm.at[pl.ds(base, CHUNK)], idx[b], semin[b])
        pltpu.async_copy(c_hbm.at[pl.ds(base, CHUNK)], cb[b], semin[b])
        pltpu.async_copy(pd4_hbm.at[wid, g], pd[b], semin[b])

    def wait_in(g, b):
        base = pl.multiple_of(wbase + g * CHUNK, CHUNK)
        pltpu.make_async_copy(row_hbm.at[pl.ds(base, CHUNK)], idx[b],
                              semin[b]).wait()
        pltpu.make_async_copy(c_hbm.at[pl.ds(base, CHUNK)], cb[b],
                              semin[b]).wait()
        pltpu.make_async_copy(pd4_hbm.at[wid, g], pd[b], semin[b]).wait()

    issue_in(0, 0)

    def pair(i, carry):
        for b in range(2):
            g = 2 * i + b
            wait_in(g, b)

            @pl.when(g + 1 < CHUNKS_PER_WORKER)
            def _():
                issue_in(g + 1, 1 - b)

            for k in range(CHUNK // L):
                sl = pl.ds(k * L, L)
                iv = idx[b][sl]
                cv = cb[b][sl]
                plsc.addupdate_scatter(accx, [iv], pd[b][0, sl] * cv)
                plsc.addupdate_scatter(accy, [iv], pd[b][1, sl] * cv)
                plsc.addupdate_scatter(accz, [iv], pd[b][2, sl] * cv)
        return carry

    lax.fori_loop(0, CHUNKS_PER_WORKER // 2, pair, 0, unroll=False)

    pltpu.sync_copy(accx, cp_hbm.at[0, wid])
    pltpu.sync_copy(accy, cp_hbm.at[1, wid])
    pltpu.sync_copy(accz, cp_hbm.at[2, wid])


def _stage4b(row_idx, c1d, pd4):
    mesh = plsc.VectorSubcoreMesh(core_axis_name="c", subcore_axis_name="s")
    f = pl.kernel(
        _coord_body,
        compiler_params=pltpu.CompilerParams(needs_layout_passes=False),
        out_type=jax.ShapeDtypeStruct((3, NW, N_PAD), jnp.float32),
        mesh=mesh,
        scratch_types=[
            pltpu.VMEM((CHUNK,), jnp.int32),
            pltpu.VMEM((CHUNK,), jnp.int32),
            pltpu.VMEM((CHUNK,), jnp.float32),
            pltpu.VMEM((CHUNK,), jnp.float32),
            pltpu.VMEM((4, CHUNK), jnp.float32),
            pltpu.VMEM((4, CHUNK), jnp.float32),
            pltpu.VMEM((N_PAD,), jnp.float32),
            pltpu.VMEM((N_PAD,), jnp.float32),
            pltpu.VMEM((N_PAD,), jnp.float32),
            pltpu.SemaphoreType.DMA,
            pltpu.SemaphoreType.DMA,
        ],
    )
    return f(row_idx, c1d, pd4)


def _stage5_body(a_ref, cp_ref, xc_ref, post_ref, wna_ref, bn_ref,
                 xo_ref, pot_ref):
    agg = a_ref[0] + a_ref[1]
    z = (xc_ref[...] + jnp.dot(agg, wna_ref[...], preferred_element_type=jnp.float32)
         + bn_ref[...])
    xo_ref[...] = z * jax.nn.sigmoid(z)
    pot_ref[...] = post_ref[...] + jnp.sum(cp_ref[...], axis=1)


def _stage5(acc2, cp, xc, pos_t, wna_t, bn_row):
    blk = 1024
    return pl.pallas_call(
        _stage5_body,
        grid=(N_PAD // blk,),
        in_specs=[
            pl.BlockSpec((2, blk, FEAT), lambda i: (0, i, 0)),
            pl.BlockSpec((3, NW, blk), lambda i: (0, 0, i)),
            pl.BlockSpec((blk, FEAT), lambda i: (i, 0)),
            pl.BlockSpec((3, blk), lambda i: (0, i)),
            pl.BlockSpec((FEAT, FEAT), lambda i: (0, 0)),
            pl.BlockSpec((1, FEAT), lambda i: (0, 0)),
        ],
        out_specs=[
            pl.BlockSpec((blk, FEAT), lambda i: (i, 0)),
            pl.BlockSpec((3, blk), lambda i: (0, i)),
        ],
        out_shape=[
            jax.ShapeDtypeStruct((N_PAD, FEAT), jnp.float32),
            jax.ShapeDtypeStruct((3, N_PAD), jnp.float32),
        ],
    )(acc2, cp, xc, pos_t, wna_t, bn_row)


def kernel(x, pos, edge_index, W_e1, b_e1, W_e2, b_e2, W_n, b_n, W_c, b_c):
    row = edge_index[0].astype(jnp.int32)
    col = edge_index[1].astype(jnp.int32)
    pad_iota = jnp.arange(E_PAD - N_EDGES, dtype=jnp.int32)
    row_pad = jnp.concatenate([row, N_NODES + pad_iota % (N_PAD - N_NODES)])
    col_pad = jnp.concatenate([col, pad_iota % N_PAD])

    x_pad = jnp.pad(x, ((0, N_PAD - N_NODES), (0, 0)))
    pos_t = jnp.pad(pos, ((0, N_PAD - N_NODES), (0, 0))).T
    px, py, pz = pos_t[0], pos_t[1], pos_t[2]

    wa_t = W_e1[:, :FEAT].T
    wb_t = W_e1[:, FEAT:2 * FEAT].T
    wd_row = W_e1[:, 2 * FEAT].reshape(1, HID)
    wnx_t = W_n[:, :FEAT].T
    wna_t = W_n[:, FEAT:].T

    t_tab, u_tab, xc = _stage1(x_pad, wa_t, wb_t, wnx_t)
    p, pd4 = _stage2(t_tab, u_tab, row_pad, col_pad, px, py, pz)
    p3 = p.reshape(E_PAD // CHUNK, CHUNK, FEAT)
    m3, c2 = _stage3(p3, pd4, wd_row,
                     b_e1.reshape(1, HID), W_e2.T.astype(jnp.bfloat16),
                     b_e2.reshape(1, HID),
                     W_c.reshape(1, HID), b_c.reshape(1, 1))
    m_rows = m3.reshape(E_PAD, FEAT)
    c1d = c2.reshape(E_PAD)
    acc2 = _stage4(m_rows, row_pad)
    cp = _stage4b(row_pad, c1d, pd4)
    x_out_pad, pos_out_t = _stage5(acc2, cp, xc, pos_t, wna_t,
                                   b_n.reshape(1, FEAT))

    return (x_out_pad[:N_NODES], pos_out_t[:, :N_NODES].T)


# speedup vs baseline: 6.6601x; 1.0869x over previous
import jax
import jax.numpy as jnp
from jax import lax
from jax.experimental import pallas as pl
from jax.experimental.pallas import tpu as pltpu
from jax.experimental.pallas import tpu_sc as plsc

N_NODES = 10000
N_EDGES = 320000
FEAT = 128
HID = 128
N_PAD = 10240
E_PAD = 327680
NW = 32
CHUNK = 128
EDGES_PER_WORKER = E_PAD // NW
CHUNKS_PER_WORKER = EDGES_PER_WORKER // CHUNK
ROWS_PER_TILE = N_PAD // 16
L = 16


def _stage1_body(x_ref, wa_ref, wb_ref, wnx_ref, t_ref, u_ref, xc_ref):
    x = x_ref[...]
    t_ref[...] = jnp.dot(x, wa_ref[...], preferred_element_type=jnp.float32)
    u_ref[...] = jnp.dot(x, wb_ref[...], preferred_element_type=jnp.float32)
    xc_ref[...] = jnp.dot(x, wnx_ref[...], preferred_element_type=jnp.float32)


def _stage1(x_pad, wa_t, wb_t, wnx_t):
    blk = 1024
    return pl.pallas_call(
        _stage1_body,
        grid=(N_PAD // blk,),
        in_specs=[
            pl.BlockSpec((blk, FEAT), lambda i: (i, 0)),
            pl.BlockSpec((FEAT, FEAT), lambda i: (0, 0)),
            pl.BlockSpec((FEAT, FEAT), lambda i: (0, 0)),
            pl.BlockSpec((FEAT, FEAT), lambda i: (0, 0)),
        ],
        out_specs=[
            pl.BlockSpec((blk, FEAT), lambda i: (i, 0)),
            pl.BlockSpec((blk, FEAT), lambda i: (i, 0)),
            pl.BlockSpec((blk, FEAT), lambda i: (i, 0)),
        ],
        out_shape=[
            jax.ShapeDtypeStruct((N_PAD, FEAT), jnp.float32),
            jax.ShapeDtypeStruct((N_PAD, FEAT), jnp.float32),
            jax.ShapeDtypeStruct((N_PAD, FEAT), jnp.float32),
        ],
    )(x_pad, wa_t, wb_t, wnx_t)


def _gather_body(t_hbm, u_hbm, row_hbm, col_hbm, px_hbm, py_hbm, pz_hbm,
                 p_hbm, pd4_hbm,
                 idx_r0, idx_r1, idx_c0, idx_c1, bt0, bt1, bu0, bu1,
                 posx, posy, posz, pd0, pd1,
                 sem_idx, sem_t, sem_u, semo0, semo1):
    c = lax.axis_index("c")
    s = lax.axis_index("s")
    wid = c * 16 + s
    wbase = wid * EDGES_PER_WORKER
    idx_r = [idx_r0, idx_r1]
    idx_c = [idx_c0, idx_c1]
    bt = [bt0, bt1]
    bu = [bu0, bu1]
    pd = [pd0, pd1]
    semo = [semo0, semo1]

    pltpu.sync_copy(px_hbm, posx)
    pltpu.sync_copy(py_hbm, posy)
    pltpu.sync_copy(pz_hbm, posz)

    def front(g, b):
        base = pl.multiple_of(wbase + g * CHUNK, CHUNK)
        pltpu.make_async_copy(row_hbm.at[pl.ds(base, CHUNK)], idx_r[b],
                              sem_idx).wait()
        pltpu.make_async_copy(col_hbm.at[pl.ds(base, CHUNK)], idx_c[b],
                              sem_idx).wait()
        pltpu.async_copy(t_hbm.at[idx_r[b]], bt[b], sem_t)
        pltpu.async_copy(u_hbm.at[idx_c[b]], bu[b], sem_u)
        for k in range(CHUNK // L):
            sl = pl.ds(k * L, L)
            ir = idx_r[b][sl]
            ic = idx_c[b][sl]
            dx = plsc.load_gather(posx, [ir]) - plsc.load_gather(posx, [ic])
            dy = plsc.load_gather(posy, [ir]) - plsc.load_gather(posy, [ic])
            dz = plsc.load_gather(posz, [ir]) - plsc.load_gather(posz, [ic])
            pd[b][0, sl] = dx
            pd[b][1, sl] = dy
            pd[b][2, sl] = dz
            pd[b][3, sl] = dx * dx + dy * dy + dz * dz
        pltpu.async_copy(pd[b], pd4_hbm.at[wid, g], semo[b])

    def back(g, b):
        base = pl.multiple_of(wbase + g * CHUNK, CHUNK)
        pltpu.make_async_copy(t_hbm.at[idx_r[b]], bt[b], sem_t).wait()
        pltpu.make_async_copy(u_hbm.at[idx_c[b]], bu[b], sem_u).wait()

        def add_row(i, carry2):
            for j in range(FEAT // L):
                sl2 = pl.ds(j * L, L)
                plsc.addupdate(bt[b].at[i, sl2], bu[b][i, sl2])
            return carry2

        lax.fori_loop(0, CHUNK, add_row, 0, unroll=False)
        pltpu.async_copy(bt[b], p_hbm.at[pl.ds(base, CHUNK)], semo[b])

    def issue_idx(g, b):
        base = pl.multiple_of(wbase + g * CHUNK, CHUNK)
        pltpu.async_copy(row_hbm.at[pl.ds(base, CHUNK)], idx_r[b], sem_idx)
        pltpu.async_copy(col_hbm.at[pl.ds(base, CHUNK)], idx_c[b], sem_idx)

    def drain_out(g, b):
        base = pl.multiple_of(wbase + g * CHUNK, CHUNK)
        pltpu.make_async_copy(pd[b], pd4_hbm.at[wid, g], semo[b]).wait()
        pltpu.make_async_copy(bt[b], p_hbm.at[pl.ds(base, CHUNK)],
                              semo[b]).wait()

    issue_idx(0, 0)
    issue_idx(1, 1)

    def pair(i, carry):
        for b in range(2):
            g = 2 * i + b

            @pl.when(g >= 2)
            def _():
                drain_out(g - 2, b)

            front(g, b)

            @pl.when(g >= 1)
            def _():
                back(g - 1, 1 - b)

            @pl.when(jnp.logical_and(g >= 1, g + 1 < CHUNKS_PER_WORKER))
            def _():
                issue_idx(g + 1, 1 - b)
        return carry

    lax.fori_loop(0, CHUNKS_PER_WORKER // 2, pair, 0, unroll=False)

    back(CHUNKS_PER_WORKER - 1, (CHUNKS_PER_WORKER - 1) % 2)
    drain_out(CHUNKS_PER_WORKER - 2, (CHUNKS_PER_WORKER - 2) % 2)
    drain_out(CHUNKS_PER_WORKER - 1, (CHUNKS_PER_WORKER - 1) % 2)


def _stage2(t_tab, u_tab, row_idx, col_idx, px, py, pz):
    mesh = plsc.VectorSubcoreMesh(core_axis_name="c", subcore_axis_name="s")
    f = pl.kernel(
        _gather_body,
        compiler_params=pltpu.CompilerParams(needs_layout_passes=False),
        out_type=[
            jax.ShapeDtypeStruct((E_PAD, FEAT), jnp.float32),
            jax.ShapeDtypeStruct((NW, CHUNKS_PER_WORKER, 4, CHUNK),
                                 jnp.float32),
        ],
        mesh=mesh,
        scratch_types=[
            pltpu.VMEM((CHUNK,), jnp.int32),
            pltpu.VMEM((CHUNK,), jnp.int32),
            pltpu.VMEM((CHUNK,), jnp.int32),
            pltpu.VMEM((CHUNK,), jnp.int32),
            pltpu.VMEM((CHUNK, FEAT), jnp.float32),
            pltpu.VMEM((CHUNK, FEAT), jnp.float32),
            pltpu.VMEM((CHUNK, FEAT), jnp.float32),
            pltpu.VMEM((CHUNK, FEAT), jnp.float32),
            pltpu.VMEM((N_PAD,), jnp.float32),
            pltpu.VMEM((N_PAD,), jnp.float32),
            pltpu.VMEM((N_PAD,), jnp.float32),
            pltpu.VMEM((4, CHUNK), jnp.float32),
            pltpu.VMEM((4, CHUNK), jnp.float32),
            pltpu.SemaphoreType.DMA,
            pltpu.SemaphoreType.DMA,
            pltpu.SemaphoreType.DMA,
            pltpu.SemaphoreType.DMA,
            pltpu.SemaphoreType.DMA,
        ],
    )
    return f(t_tab, u_tab, row_idx, col_idx, px, py, pz)


def _stage3_body(p_ref, pd4_ref, wd_ref, b1_ref, we2_ref, b2_ref, wc_ref,
                 bc_ref, m_ref, c_ref):
    wd = wd_ref[...]
    b1 = b1_ref[...]
    d2t = jnp.transpose(pd4_ref[0, :, 3, :], (1, 0))
    p = p_ref[...]
    parts = [p[i * CHUNK:(i + 1) * CHUNK, :] + d2t[:, i:i + 1] * wd
             for i in range(8)]
    pre = jnp.concatenate(parts, axis=0) + b1
    h = pre * jax.nn.sigmoid(pre)
    z = jnp.dot(h.astype(jnp.bfloat16), we2_ref[...],
                preferred_element_type=jnp.float32) + b2_ref[...]
    m = z * jax.nn.sigmoid(z)
    m_ref[...] = m
    wc = wc_ref[...]
    bc = bc_ref[...]
    for i in range(8):
        cz = lax.dot_general(wc, m[i * CHUNK:(i + 1) * CHUNK, :],
                             (((1,), (1,)), ((), ())),
                             preferred_element_type=jnp.float32)
        c_ref[pl.ds(i, 1), :] = jnp.tanh(cz + bc)


def _stage3(p, pd4, wd_row, b1_row, we2_bf, b2_row, wc_row, bc_s):
    blk = 8 * CHUNK
    nblk = E_PAD // blk
    gpw = CHUNKS_PER_WORKER // 8
    return pl.pallas_call(
        _stage3_body,
        grid=(nblk,),
        in_specs=[
            pl.BlockSpec((blk, FEAT), lambda i: (i, 0)),
            pl.BlockSpec((1, 8, 4, CHUNK), lambda i: (i // gpw, i % gpw, 0, 0)),
            pl.BlockSpec((1, FEAT), lambda i: (0, 0)),
            pl.BlockSpec((1, FEAT), lambda i: (0, 0)),
            pl.BlockSpec((FEAT, FEAT), lambda i: (0, 0)),
            pl.BlockSpec((1, FEAT), lambda i: (0, 0)),
            pl.BlockSpec((1, FEAT), lambda i: (0, 0)),
            pl.BlockSpec((1, 1), lambda i: (0, 0)),
        ],
        out_specs=[
            pl.BlockSpec((blk, FEAT), lambda i: (i, 0)),
            pl.BlockSpec((8, CHUNK), lambda i: (i, 0)),
        ],
        out_shape=[
            jax.ShapeDtypeStruct((E_PAD, FEAT), jnp.float32),
            jax.ShapeDtypeStruct((E_PAD // CHUNK, CHUNK), jnp.float32),
        ],
    )(p, pd4, wd_row, b1_row, we2_bf, b2_row, wc_row, bc_s)


def _scatter_body(m_hbm, row_hbm, out_hbm,
                  acc, m0, m1, idx0, idx1, semin0, semin1, semsc0, semsc1):
    c = lax.axis_index("c")
    s = lax.axis_index("s")
    wid = c * 16 + s
    wbase = wid * EDGES_PER_WORKER
    mb = [m0, m1]
    idx = [idx0, idx1]
    semin = [semin0, semin1]
    semsc = [semsc0, semsc1]

    def zrow(i, carry2):
        for j in range(FEAT // L):
            m0[i, pl.ds(j * L, L)] = jnp.zeros((L,), jnp.float32)
        return carry2

    lax.fori_loop(0, CHUNK, zrow, 0, unroll=False)

    def zcopy(k, carry2):
        pltpu.sync_copy(m0, acc.at[pl.ds(s * ROWS_PER_TILE + k * CHUNK, CHUNK)])
        return carry2

    lax.fori_loop(0, ROWS_PER_TILE // CHUNK, zcopy, 0, unroll=False)
    plsc.subcore_barrier()

    def issue_in(g, b):
        base = pl.multiple_of(wbase + g * CHUNK, CHUNK)
        pltpu.async_copy(row_hbm.at[pl.ds(base, CHUNK)], idx[b], semin[b])
        pltpu.async_copy(m_hbm.at[pl.ds(base, CHUNK)], mb[b], semin[b])

    def wait_in(g, b):
        base = pl.multiple_of(wbase + g * CHUNK, CHUNK)
        pltpu.make_async_copy(row_hbm.at[pl.ds(base, CHUNK)], idx[b],
                              semin[b]).wait()
        pltpu.make_async_copy(m_hbm.at[pl.ds(base, CHUNK)], mb[b],
                              semin[b]).wait()

    issue_in(0, 0)

    def pair(i, carry):
        for b in range(2):
            g = 2 * i + b
            wait_in(g, b)

            @pl.when(g >= 1)
            def _():
                pltpu.make_async_copy(mb[1 - b], acc.at[idx[1 - b]],
                                      semsc[1 - b]).wait()

            @pl.when(g + 1 < CHUNKS_PER_WORKER)
            def _():
                issue_in(g + 1, 1 - b)

            pltpu.async_copy(mb[b], acc.at[idx[b]], semsc[b], add=True)
        return carry

    lax.fori_loop(0, CHUNKS_PER_WORKER // 2, pair, 0, unroll=False)
    lastb = (CHUNKS_PER_WORKER - 1) % 2
    pltpu.make_async_copy(mb[lastb], acc.at[idx[lastb]], semsc[lastb]).wait()
    plsc.subcore_barrier()

    pltpu.sync_copy(acc.at[pl.ds(s * ROWS_PER_TILE, ROWS_PER_TILE)],
                    out_hbm.at[c, pl.ds(s * ROWS_PER_TILE, ROWS_PER_TILE)])


def _stage4(m_rows, row_idx):
    mesh = plsc.VectorSubcoreMesh(core_axis_name="c", subcore_axis_name="s")
    f = pl.kernel(
        _scatter_body,
        compiler_params=pltpu.CompilerParams(needs_layout_passes=False),
        out_type=jax.ShapeDtypeStruct((2, N_PAD, FEAT), jnp.float32),
        mesh=mesh,
        scratch_types=[
            pltpu.VMEM_SHARED((N_PAD, FEAT), jnp.float32),
            pltpu.VMEM((CHUNK, FEAT), jnp.float32),
            pltpu.VMEM((CHUNK, FEAT), jnp.float32),
            pltpu.VMEM((CHUNK,), jnp.int32),
            pltpu.VMEM((CHUNK,), jnp.int32),
            pltpu.SemaphoreType.DMA,
            pltpu.SemaphoreType.DMA,
            pltpu.SemaphoreType.DMA,
            pltpu.SemaphoreType.DMA,
        ],
    )
    return f(m_rows, row_idx)


def _coord_body(row_hbm, c_hbm, pd4_hbm, cp_hbm,
                idx0, idx1, c0, c1, pd0, pd1,
                accx, accy, accz, semin0, semin1):
    c = lax.axis_index("c")
    s = lax.axis_index("s")
    wid = c * 16 + s
    wbase = wid * EDGES_PER_WORKER
    idx = [idx0, idx1]
    cb = [c0, c1]
    pd = [pd0, pd1]
    semin = [semin0, semin1]

    def zacc(k, carry2):
        sl = pl.ds(k * L, L)
        z = jnp.zeros((L,), jnp.float32)
        accx[sl] = z
        accy[sl] = z
        accz[sl] = z
        return carry2

    lax.fori_loop(0, N_PAD // L, zacc, 0, unroll=False)

    def issue_in(g, b):
        base = pl.multiple_of(wbase + g * CHUNK, CHUNK)
        pltpu.async_copy(row_hbm.at[pl.ds(base, CHUNK)], idx[b], semin[b])
        pltpu.async_copy(c_hbm.at[pl.ds(base, CHUNK)], cb[b], semin[b])
        pltpu.async_copy(pd4_hbm.at[wid, g], pd[b], semin[b])

    def wait_in(g, b):
        base = pl.multiple_of(wbase + g * CHUNK, CHUNK)
        pltpu.make_async_copy(row_hbm.at[pl.ds(base, CHUNK)], idx[b],
                              semin[b]).wait()
        pltpu.make_async_copy(c_hbm.at[pl.ds(base, CHUNK)], cb[b],
                              semin[b]).wait()
        pltpu.make_async_copy(pd4_hbm.at[wid, g], pd[b], semin[b]).wait()

    issue_in(0, 0)

    def pair(i, carry):
        for b in range(2):
            g = 2 * i + b
            wait_in(g, b)

            @pl.when(g + 1 < CHUNKS_PER_WORKER)
            def _():
                issue_in(g + 1, 1 - b)

            for k in range(CHUNK // L):
                sl = pl.ds(k * L, L)
                iv = idx[b][sl]
                cv = cb[b][sl]
                plsc.addupdate_scatter(accx, [iv], pd[b][0, sl] * cv)
                plsc.addupdate_scatter(accy, [iv], pd[b][1, sl] * cv)
                plsc.addupdate_scatter(accz, [iv], pd[b][2, sl] * cv)
        return carry

    lax.fori_loop(0, CHUNKS_PER_WORKER // 2, pair, 0, unroll=False)

    pltpu.sync_copy(accx, cp_hbm.at[0, wid])
    pltpu.sync_copy(accy, cp_hbm.at[1, wid])
    pltpu.sync_copy(accz, cp_hbm.at[2, wid])


def _stage4b(row_idx, c1d, pd4):
    mesh = plsc.VectorSubcoreMesh(core_axis_name="c", subcore_axis_name="s")
    f = pl.kernel(
        _coord_body,
        compiler_params=pltpu.CompilerParams(needs_layout_passes=False),
        out_type=jax.ShapeDtypeStruct((3, NW, N_PAD), jnp.float32),
        mesh=mesh,
        scratch_types=[
            pltpu.VMEM((CHUNK,), jnp.int32),
            pltpu.VMEM((CHUNK,), jnp.int32),
            pltpu.VMEM((CHUNK,), jnp.float32),
            pltpu.VMEM((CHUNK,), jnp.float32),
            pltpu.VMEM((4, CHUNK), jnp.float32),
            pltpu.VMEM((4, CHUNK), jnp.float32),
            pltpu.VMEM((N_PAD,), jnp.float32),
            pltpu.VMEM((N_PAD,), jnp.float32),
            pltpu.VMEM((N_PAD,), jnp.float32),
            pltpu.SemaphoreType.DMA,
            pltpu.SemaphoreType.DMA,
        ],
    )
    return f(row_idx, c1d, pd4)


def _stage5_body(a_ref, cp_ref, xc_ref, post_ref, wna_ref, bn_ref,
                 xo_ref, pot_ref):
    agg = a_ref[0] + a_ref[1]
    z = (xc_ref[...] + jnp.dot(agg, wna_ref[...], preferred_element_type=jnp.float32)
         + bn_ref[...])
    xo_ref[...] = z * jax.nn.sigmoid(z)
    pot_ref[...] = post_ref[...] + jnp.sum(cp_ref[...], axis=1)


def _stage5(acc2, cp, xc, pos_t, wna_t, bn_row):
    blk = 1024
    return pl.pallas_call(
        _stage5_body,
        grid=(N_PAD // blk,),
        in_specs=[
            pl.BlockSpec((2, blk, FEAT), lambda i: (0, i, 0)),
            pl.BlockSpec((3, NW, blk), lambda i: (0, 0, i)),
            pl.BlockSpec((blk, FEAT), lambda i: (i, 0)),
            pl.BlockSpec((3, blk), lambda i: (0, i)),
            pl.BlockSpec((FEAT, FEAT), lambda i: (0, 0)),
            pl.BlockSpec((1, FEAT), lambda i: (0, 0)),
        ],
        out_specs=[
            pl.BlockSpec((blk, FEAT), lambda i: (i, 0)),
            pl.BlockSpec((3, blk), lambda i: (0, i)),
        ],
        out_shape=[
            jax.ShapeDtypeStruct((N_PAD, FEAT), jnp.float32),
            jax.ShapeDtypeStruct((3, N_PAD), jnp.float32),
        ],
    )(acc2, cp, xc, pos_t, wna_t, bn_row)


def kernel(x, pos, edge_index, W_e1, b_e1, W_e2, b_e2, W_n, b_n, W_c, b_c):
    row = edge_index[0].astype(jnp.int32)
    col = edge_index[1].astype(jnp.int32)
    pad_iota = jnp.arange(E_PAD - N_EDGES, dtype=jnp.int32)
    row_pad = jnp.concatenate([row, N_NODES + pad_iota % (N_PAD - N_NODES)])
    col_pad = jnp.concatenate([col, pad_iota % N_PAD])

    x_pad = jnp.pad(x, ((0, N_PAD - N_NODES), (0, 0)))
    pos_t = jnp.pad(pos, ((0, N_PAD - N_NODES), (0, 0))).T
    px, py, pz = pos_t[0], pos_t[1], pos_t[2]

    wa_t = W_e1[:, :FEAT].T
    wb_t = W_e1[:, FEAT:2 * FEAT].T
    wd_row = W_e1[:, 2 * FEAT].reshape(1, HID)
    wnx_t = W_n[:, :FEAT].T
    wna_t = W_n[:, FEAT:].T

    t_tab, u_tab, xc = _stage1(x_pad, wa_t, wb_t, wnx_t)
    p, pd4 = _stage2(t_tab, u_tab, row_pad, col_pad, px, py, pz)
    m_rows, c2 = _stage3(p, pd4, wd_row,
                         b_e1.reshape(1, HID), W_e2.T.astype(jnp.bfloat16),
                         b_e2.reshape(1, HID),
                         W_c.reshape(1, HID), b_c.reshape(1, 1))
    c1d = c2.reshape(E_PAD)
    acc2 = _stage4(m_rows, row_pad)
    cp = _stage4b(row_pad, c1d, pd4)
    x_out_pad, pos_out_t = _stage5(acc2, cp, xc, pos_t, wna_t,
                                   b_n.reshape(1, FEAT))

    return (x_out_pad[:N_NODES], pos_out_t[:, :N_NODES].T)


# speedup vs baseline: 7.8933x; 1.1852x over previous
import jax
import jax.numpy as jnp
from jax import lax
from jax.experimental import pallas as pl
from jax.experimental.pallas import tpu as pltpu
from jax.experimental.pallas import tpu_sc as plsc

N_NODES = 10000
N_EDGES = 320000
FEAT = 128
HID = 128
N_PAD = 10240
E_PAD = 327680
N_SLICES = 2
E_SLICE = E_PAD // N_SLICES
NW = 32
CHUNK = 128
EDGES_PER_WORKER = E_SLICE // NW
CHUNKS_PER_WORKER = EDGES_PER_WORKER // CHUNK
ROWS_PER_TILE = N_PAD // 16
L = 16


def _stage1_body(x_ref, wa_ref, wb_ref, wnx_ref, t_ref, u_ref, xc_ref):
    x = x_ref[...]
    t_ref[...] = jnp.dot(x, wa_ref[...], preferred_element_type=jnp.float32)
    u_ref[...] = jnp.dot(x, wb_ref[...], preferred_element_type=jnp.float32)
    xc_ref[...] = jnp.dot(x, wnx_ref[...], preferred_element_type=jnp.float32)


def _stage1(x_pad, wa_t, wb_t, wnx_t):
    blk = 1024
    return pl.pallas_call(
        _stage1_body,
        grid=(N_PAD // blk,),
        in_specs=[
            pl.BlockSpec((blk, FEAT), lambda i: (i, 0)),
            pl.BlockSpec((FEAT, FEAT), lambda i: (0, 0)),
            pl.BlockSpec((FEAT, FEAT), lambda i: (0, 0)),
            pl.BlockSpec((FEAT, FEAT), lambda i: (0, 0)),
        ],
        out_specs=[
            pl.BlockSpec((blk, FEAT), lambda i: (i, 0)),
            pl.BlockSpec((blk, FEAT), lambda i: (i, 0)),
            pl.BlockSpec((blk, FEAT), lambda i: (i, 0)),
        ],
        out_shape=[
            jax.ShapeDtypeStruct((N_PAD, FEAT), jnp.float32),
            jax.ShapeDtypeStruct((N_PAD, FEAT), jnp.float32),
            jax.ShapeDtypeStruct((N_PAD, FEAT), jnp.float32),
        ],
    )(x_pad, wa_t, wb_t, wnx_t)


def _gather_body(t_hbm, u_hbm, row_hbm, col_hbm, px_hbm, py_hbm, pz_hbm,
                 p_hbm, pd4_hbm,
                 idx_r0, idx_r1, idx_c0, idx_c1, bt0, bt1, bu0, bu1,
                 posx, posy, posz, pd0, pd1,
                 sem_idx, sem_t, sem_u, semo0, semo1):
    c = lax.axis_index("c")
    s = lax.axis_index("s")
    wid = c * 16 + s
    wbase = wid * EDGES_PER_WORKER
    idx_r = [idx_r0, idx_r1]
    idx_c = [idx_c0, idx_c1]
    bt = [bt0, bt1]
    bu = [bu0, bu1]
    pd = [pd0, pd1]
    semo = [semo0, semo1]

    pltpu.sync_copy(px_hbm, posx)
    pltpu.sync_copy(py_hbm, posy)
    pltpu.sync_copy(pz_hbm, posz)

    def front(g, b):
        base = pl.multiple_of(wbase + g * CHUNK, CHUNK)
        pltpu.make_async_copy(row_hbm.at[pl.ds(base, CHUNK)], idx_r[b],
                              sem_idx).wait()
        pltpu.make_async_copy(col_hbm.at[pl.ds(base, CHUNK)], idx_c[b],
                              sem_idx).wait()
        pltpu.async_copy(t_hbm.at[idx_r[b]], bt[b], sem_t)
        pltpu.async_copy(u_hbm.at[idx_c[b]], bu[b], sem_u)
        for k in range(CHUNK // L):
            sl = pl.ds(k * L, L)
            ir = idx_r[b][sl]
            ic = idx_c[b][sl]
            dx = plsc.load_gather(posx, [ir]) - plsc.load_gather(posx, [ic])
            dy = plsc.load_gather(posy, [ir]) - plsc.load_gather(posy, [ic])
            dz = plsc.load_gather(posz, [ir]) - plsc.load_gather(posz, [ic])
            pd[b][0, sl] = dx
            pd[b][1, sl] = dy
            pd[b][2, sl] = dz
            pd[b][3, sl] = dx * dx + dy * dy + dz * dz
        pltpu.async_copy(pd[b], pd4_hbm.at[wid, g], semo[b])

    def back(g, b):
        base = pl.multiple_of(wbase + g * CHUNK, CHUNK)
        pltpu.make_async_copy(t_hbm.at[idx_r[b]], bt[b], sem_t).wait()
        pltpu.make_async_copy(u_hbm.at[idx_c[b]], bu[b], sem_u).wait()

        def add_row(i, carry2):
            for j in range(FEAT // L):
                sl2 = pl.ds(j * L, L)
                plsc.addupdate(bt[b].at[i, sl2], bu[b][i, sl2])
            return carry2

        lax.fori_loop(0, CHUNK, add_row, 0, unroll=False)
        pltpu.async_copy(bt[b], p_hbm.at[pl.ds(base, CHUNK)], semo[b])

    def issue_idx(g, b):
        base = pl.multiple_of(wbase + g * CHUNK, CHUNK)
        pltpu.async_copy(row_hbm.at[pl.ds(base, CHUNK)], idx_r[b], sem_idx)
        pltpu.async_copy(col_hbm.at[pl.ds(base, CHUNK)], idx_c[b], sem_idx)

    def drain_out(g, b):
        base = pl.multiple_of(wbase + g * CHUNK, CHUNK)
        pltpu.make_async_copy(pd[b], pd4_hbm.at[wid, g], semo[b]).wait()
        pltpu.make_async_copy(bt[b], p_hbm.at[pl.ds(base, CHUNK)],
                              semo[b]).wait()

    issue_idx(0, 0)
    issue_idx(1, 1)

    def pair(i, carry):
        for b in range(2):
            g = 2 * i + b

            @pl.when(g >= 2)
            def _():
                drain_out(g - 2, b)

            front(g, b)

            @pl.when(g >= 1)
            def _():
                back(g - 1, 1 - b)

            @pl.when(jnp.logical_and(g >= 1, g + 1 < CHUNKS_PER_WORKER))
            def _():
                issue_idx(g + 1, 1 - b)
        return carry

    lax.fori_loop(0, CHUNKS_PER_WORKER // 2, pair, 0, unroll=False)

    back(CHUNKS_PER_WORKER - 1, (CHUNKS_PER_WORKER - 1) % 2)
    drain_out(CHUNKS_PER_WORKER - 2, (CHUNKS_PER_WORKER - 2) % 2)
    drain_out(CHUNKS_PER_WORKER - 1, (CHUNKS_PER_WORKER - 1) % 2)


def _stage2(t_tab, u_tab, row_idx, col_idx, px, py, pz):
    mesh = plsc.VectorSubcoreMesh(core_axis_name="c", subcore_axis_name="s")
    f = pl.kernel(
        _gather_body,
        compiler_params=pltpu.CompilerParams(needs_layout_passes=False),
        out_type=[
            jax.ShapeDtypeStruct((E_SLICE, FEAT), jnp.float32),
            jax.ShapeDtypeStruct((NW, CHUNKS_PER_WORKER, 4, CHUNK),
                                 jnp.float32),
        ],
        mesh=mesh,
        scratch_types=[
            pltpu.VMEM((CHUNK,), jnp.int32),
            pltpu.VMEM((CHUNK,), jnp.int32),
            pltpu.VMEM((CHUNK,), jnp.int32),
            pltpu.VMEM((CHUNK,), jnp.int32),
            pltpu.VMEM((CHUNK, FEAT), jnp.float32),
            pltpu.VMEM((CHUNK, FEAT), jnp.float32),
            pltpu.VMEM((CHUNK, FEAT), jnp.float32),
            pltpu.VMEM((CHUNK, FEAT), jnp.float32),
            pltpu.VMEM((N_PAD,), jnp.float32),
            pltpu.VMEM((N_PAD,), jnp.float32),
            pltpu.VMEM((N_PAD,), jnp.float32),
            pltpu.VMEM((4, CHUNK), jnp.float32),
            pltpu.VMEM((4, CHUNK), jnp.float32),
            pltpu.SemaphoreType.DMA,
            pltpu.SemaphoreType.DMA,
            pltpu.SemaphoreType.DMA,
            pltpu.SemaphoreType.DMA,
            pltpu.SemaphoreType.DMA,
        ],
    )
    return f(t_tab, u_tab, row_idx, col_idx, px, py, pz)


def _stage3_body(p_ref, pd4_ref, wd_ref, b1_ref, we2_ref, b2_ref, wc_ref,
                 bc_ref, m_ref, c_ref):
    wd = wd_ref[...]
    b1 = b1_ref[...]
    d2t = jnp.transpose(pd4_ref[0, :, 3, :], (1, 0))
    p = p_ref[...]
    parts = [p[i * CHUNK:(i + 1) * CHUNK, :] + d2t[:, i:i + 1] * wd
             for i in range(8)]
    pre = jnp.concatenate(parts, axis=0) + b1
    h = pre * jax.nn.sigmoid(pre)
    z = jnp.dot(h.astype(jnp.bfloat16), we2_ref[...],
                preferred_element_type=jnp.float32) + b2_ref[...]
    m = z * jax.nn.sigmoid(z)
    m_ref[...] = m
    wc = wc_ref[...]
    bc = bc_ref[...]
    for i in range(8):
        cz = lax.dot_general(wc, m[i * CHUNK:(i + 1) * CHUNK, :],
                             (((1,), (1,)), ((), ())),
                             preferred_element_type=jnp.float32)
        c_ref[pl.ds(i, 1), :] = jnp.tanh(cz + bc)


def _stage3(p, pd4, wd_row, b1_row, we2_bf, b2_row, wc_row, bc_s):
    blk = 8 * CHUNK
    nblk = E_SLICE // blk
    gpw = CHUNKS_PER_WORKER // 8
    return pl.pallas_call(
        _stage3_body,
        grid=(nblk,),
        in_specs=[
            pl.BlockSpec((blk, FEAT), lambda i: (i, 0)),
            pl.BlockSpec((1, 8, 4, CHUNK), lambda i: (i // gpw, i % gpw, 0, 0)),
            pl.BlockSpec((1, FEAT), lambda i: (0, 0)),
            pl.BlockSpec((1, FEAT), lambda i: (0, 0)),
            pl.BlockSpec((FEAT, FEAT), lambda i: (0, 0)),
            pl.BlockSpec((1, FEAT), lambda i: (0, 0)),
            pl.BlockSpec((1, FEAT), lambda i: (0, 0)),
            pl.BlockSpec((1, 1), lambda i: (0, 0)),
        ],
        out_specs=[
            pl.BlockSpec((blk, FEAT), lambda i: (i, 0)),
            pl.BlockSpec((8, CHUNK), lambda i: (i, 0)),
        ],
        out_shape=[
            jax.ShapeDtypeStruct((E_SLICE, FEAT), jnp.float32),
            jax.ShapeDtypeStruct((E_SLICE // CHUNK, CHUNK), jnp.float32),
        ],
    )(p, pd4, wd_row, b1_row, we2_bf, b2_row, wc_row, bc_s)


def _scatter_body(m_hbm, row_hbm, out_hbm,
                  acc, m0, m1, idx0, idx1, semin0, semin1, semsc0, semsc1):
    c = lax.axis_index("c")
    s = lax.axis_index("s")
    wid = c * 16 + s
    wbase = wid * EDGES_PER_WORKER
    mb = [m0, m1]
    idx = [idx0, idx1]
    semin = [semin0, semin1]
    semsc = [semsc0, semsc1]

    def zrow(i, carry2):
        for j in range(FEAT // L):
            m0[i, pl.ds(j * L, L)] = jnp.zeros((L,), jnp.float32)
        return carry2

    lax.fori_loop(0, CHUNK, zrow, 0, unroll=False)

    def zcopy(k, carry2):
        pltpu.sync_copy(m0, acc.at[pl.ds(s * ROWS_PER_TILE + k * CHUNK, CHUNK)])
        return carry2

    lax.fori_loop(0, ROWS_PER_TILE // CHUNK, zcopy, 0, unroll=False)
    plsc.subcore_barrier()

    def issue_in(g, b):
        base = pl.multiple_of(wbase + g * CHUNK, CHUNK)
        pltpu.async_copy(row_hbm.at[pl.ds(base, CHUNK)], idx[b], semin[b])
        pltpu.async_copy(m_hbm.at[pl.ds(base, CHUNK)], mb[b], semin[b])

    def wait_in(g, b):
        base = pl.multiple_of(wbase + g * CHUNK, CHUNK)
        pltpu.make_async_copy(row_hbm.at[pl.ds(base, CHUNK)], idx[b],
                              semin[b]).wait()
        pltpu.make_async_copy(m_hbm.at[pl.ds(base, CHUNK)], mb[b],
                              semin[b]).wait()

    issue_in(0, 0)

    def pair(i, carry):
        for b in range(2):
            g = 2 * i + b
            wait_in(g, b)

            @pl.when(g >= 1)
            def _():
                pltpu.make_async_copy(mb[1 - b], acc.at[idx[1 - b]],
                                      semsc[1 - b]).wait()

            @pl.when(g + 1 < CHUNKS_PER_WORKER)
            def _():
                issue_in(g + 1, 1 - b)

            pltpu.async_copy(mb[b], acc.at[idx[b]], semsc[b], add=True)
        return carry

    lax.fori_loop(0, CHUNKS_PER_WORKER // 2, pair, 0, unroll=False)
    lastb = (CHUNKS_PER_WORKER - 1) % 2
    pltpu.make_async_copy(mb[lastb], acc.at[idx[lastb]], semsc[lastb]).wait()
    plsc.subcore_barrier()

    pltpu.sync_copy(acc.at[pl.ds(s * ROWS_PER_TILE, ROWS_PER_TILE)],
                    out_hbm.at[c, pl.ds(s * ROWS_PER_TILE, ROWS_PER_TILE)])


def _stage4(m_rows, row_idx):
    mesh = plsc.VectorSubcoreMesh(core_axis_name="c", subcore_axis_name="s")
    f = pl.kernel(
        _scatter_body,
        compiler_params=pltpu.CompilerParams(needs_layout_passes=False),
        out_type=jax.ShapeDtypeStruct((2, N_PAD, FEAT), jnp.float32),
        mesh=mesh,
        scratch_types=[
            pltpu.VMEM_SHARED((N_PAD, FEAT), jnp.float32),
            pltpu.VMEM((CHUNK, FEAT), jnp.float32),
            pltpu.VMEM((CHUNK, FEAT), jnp.float32),
            pltpu.VMEM((CHUNK,), jnp.int32),
            pltpu.VMEM((CHUNK,), jnp.int32),
            pltpu.SemaphoreType.DMA,
            pltpu.SemaphoreType.DMA,
            pltpu.SemaphoreType.DMA,
            pltpu.SemaphoreType.DMA,
        ],
    )
    return f(m_rows, row_idx)


def _coord_body(row_hbm, c_hbm, pd4_hbm, cp_hbm,
                idx0, idx1, c0, c1, pd0, pd1,
                accx, accy, accz, semin0, semin1):
    c = lax.axis_index("c")
    s = lax.axis_index("s")
    wid = c * 16 + s
    wbase = wid * EDGES_PER_WORKER
    idx = [idx0, idx1]
    cb = [c0, c1]
    pd = [pd0, pd1]
    semin = [semin0, semin1]

    def zacc(k, carry2):
        sl = pl.ds(k * L, L)
        z = jnp.zeros((L,), jnp.float32)
        accx[sl] = z
        accy[sl] = z
        accz[sl] = z
        return carry2

    lax.fori_loop(0, N_PAD // L, zacc, 0, unroll=False)

    def issue_in(g, b):
        base = pl.multiple_of(wbase + g * CHUNK, CHUNK)
        pltpu.async_copy(row_hbm.at[pl.ds(base, CHUNK)], idx[b], semin[b])
        pltpu.async_copy(c_hbm.at[pl.ds(base, CHUNK)], cb[b], semin[b])
        pltpu.async_copy(pd4_hbm.at[wid, g], pd[b], semin[b])

    def wait_in(g, b):
        base = pl.multiple_of(wbase + g * CHUNK, CHUNK)
        pltpu.make_async_copy(row_hbm.at[pl.ds(base, CHUNK)], idx[b],
                              semin[b]).wait()
        pltpu.make_async_copy(c_hbm.at[pl.ds(base, CHUNK)], cb[b],
                              semin[b]).wait()
        pltpu.make_async_copy(pd4_hbm.at[wid, g], pd[b], semin[b]).wait()

    issue_in(0, 0)

    def pair(i, carry):
        for b in range(2):
            g = 2 * i + b
            wait_in(g, b)

            @pl.when(g + 1 < CHUNKS_PER_WORKER)
            def _():
                issue_in(g + 1, 1 - b)

            for k in range(CHUNK // L):
                sl = pl.ds(k * L, L)
                iv = idx[b][sl]
                cv = cb[b][sl]
                plsc.addupdate_scatter(accx, [iv], pd[b][0, sl] * cv)
                plsc.addupdate_scatter(accy, [iv], pd[b][1, sl] * cv)
                plsc.addupdate_scatter(accz, [iv], pd[b][2, sl] * cv)
        return carry

    lax.fori_loop(0, CHUNKS_PER_WORKER // 2, pair, 0, unroll=False)

    pltpu.sync_copy(accx, cp_hbm.at[0, wid])
    pltpu.sync_copy(accy, cp_hbm.at[1, wid])
    pltpu.sync_copy(accz, cp_hbm.at[2, wid])


def _stage4b(row_idx, c1d, pd4):
    mesh = plsc.VectorSubcoreMesh(core_axis_name="c", subcore_axis_name="s")
    f = pl.kernel(
        _coord_body,
        compiler_params=pltpu.CompilerParams(needs_layout_passes=False),
        out_type=jax.ShapeDtypeStruct((3, NW, N_PAD), jnp.float32),
        mesh=mesh,
        scratch_types=[
            pltpu.VMEM((CHUNK,), jnp.int32),
            pltpu.VMEM((CHUNK,), jnp.int32),
            pltpu.VMEM((CHUNK,), jnp.float32),
            pltpu.VMEM((CHUNK,), jnp.float32),
            pltpu.VMEM((4, CHUNK), jnp.float32),
            pltpu.VMEM((4, CHUNK), jnp.float32),
            pltpu.VMEM((N_PAD,), jnp.float32),
            pltpu.VMEM((N_PAD,), jnp.float32),
            pltpu.VMEM((N_PAD,), jnp.float32),
            pltpu.SemaphoreType.DMA,
            pltpu.SemaphoreType.DMA,
        ],
    )
    return f(row_idx, c1d, pd4)


def _stage5_body(a0_ref, a1_ref, cp0_ref, cp1_ref, xc_ref, post_ref,
                 wna_ref, bn_ref, xo_ref, pot_ref):
    agg = a0_ref[0] + a0_ref[1] + a1_ref[0] + a1_ref[1]
    z = (xc_ref[...] + jnp.dot(agg, wna_ref[...], preferred_element_type=jnp.float32)
         + bn_ref[...])
    xo_ref[...] = z * jax.nn.sigmoid(z)
    pot_ref[...] = (post_ref[...] + jnp.sum(cp0_ref[...], axis=1)
                    + jnp.sum(cp1_ref[...], axis=1))


def _stage5(acc_a, acc_b, cp_a, cp_b, xc, pos_t, wna_t, bn_row):
    blk = 1024
    acc_spec = pl.BlockSpec((2, blk, FEAT), lambda i: (0, i, 0))
    cp_spec = pl.BlockSpec((3, NW, blk), lambda i: (0, 0, i))
    return pl.pallas_call(
        _stage5_body,
        grid=(N_PAD // blk,),
        in_specs=[
            acc_spec,
            acc_spec,
            cp_spec,
            cp_spec,
            pl.BlockSpec((blk, FEAT), lambda i: (i, 0)),
            pl.BlockSpec((3, blk), lambda i: (0, i)),
            pl.BlockSpec((FEAT, FEAT), lambda i: (0, 0)),
            pl.BlockSpec((1, FEAT), lambda i: (0, 0)),
        ],
        out_specs=[
            pl.BlockSpec((blk, FEAT), lambda i: (i, 0)),
            pl.BlockSpec((3, blk), lambda i: (0, i)),
        ],
        out_shape=[
            jax.ShapeDtypeStruct((N_PAD, FEAT), jnp.float32),
            jax.ShapeDtypeStruct((3, N_PAD), jnp.float32),
        ],
    )(acc_a, acc_b, cp_a, cp_b, xc, pos_t, wna_t, bn_row)


def kernel(x, pos, edge_index, W_e1, b_e1, W_e2, b_e2, W_n, b_n, W_c, b_c):
    row = edge_index[0].astype(jnp.int32)
    col = edge_index[1].astype(jnp.int32)
    pad_iota = jnp.arange(E_PAD - N_EDGES, dtype=jnp.int32)
    row_pad = jnp.concatenate([row, N_NODES + pad_iota % (N_PAD - N_NODES)])
    col_pad = jnp.concatenate([col, pad_iota % N_PAD])

    x_pad = jnp.pad(x, ((0, N_PAD - N_NODES), (0, 0)))
    pos_t = jnp.pad(pos, ((0, N_PAD - N_NODES), (0, 0))).T
    px, py, pz = pos_t[0], pos_t[1], pos_t[2]

    wa_t = W_e1[:, :FEAT].T
    wb_t = W_e1[:, FEAT:2 * FEAT].T
    wd_row = W_e1[:, 2 * FEAT].reshape(1, HID)
    wnx_t = W_n[:, :FEAT].T
    wna_t = W_n[:, FEAT:].T

    t_tab, u_tab, xc = _stage1(x_pad, wa_t, wb_t, wnx_t)

    b1r = b_e1.reshape(1, HID)
    we2_bf = W_e2.T.astype(jnp.bfloat16)
    b2r = b_e2.reshape(1, HID)
    wcr = W_c.reshape(1, HID)
    bcr = b_c.reshape(1, 1)

    rows = [row_pad[:E_SLICE], row_pad[E_SLICE:]]
    cols = [col_pad[:E_SLICE], col_pad[E_SLICE:]]
    p_s, pd4_s, m_s, c_s = [None, None], [None, None], [None, None], [None, None]
    for s in range(N_SLICES):
        p_s[s], pd4_s[s] = _stage2(t_tab, u_tab, rows[s], cols[s], px, py, pz)
    for s in range(N_SLICES):
        m_s[s], c2 = _stage3(p_s[s], pd4_s[s], wd_row, b1r, we2_bf, b2r,
                             wcr, bcr)
        c_s[s] = c2.reshape(E_SLICE)
    acc_a = _stage4(m_s[0], rows[0])
    acc_b = _stage4(m_s[1], rows[1])
    cp_a = _stage4b(rows[0], c_s[0], pd4_s[0])
    cp_b = _stage4b(rows[1], c_s[1], pd4_s[1])
    x_out_pad, pos_out_t = _stage5(acc_a, acc_b, cp_a, cp_b, xc, pos_t, wna_t,
                                   b_n.reshape(1, FEAT))

    return (x_out_pad[:N_NODES], pos_out_t[:, :N_NODES].T)
